# Initial kernel scaffold; baseline (speedup 1.0000x reference)
import functools
import math

import jax
import jax.numpy as jnp
from jax import lax
from jax.experimental import pallas as pl
from jax.experimental.pallas import tpu as pltpu
from jax.experimental.pallas import tpu_sc as plsc

N = 10000
E = 320000
D = 128
H = 8
DH = 16
ED = 16
NORM = 1.0 / math.sqrt(DH)

NC = 2
NS = 16
NW = NC * NS
EPW = E // NW
NEG = -3.4028235e38

f32 = jnp.float32
i32 = jnp.int32


def _proj_body(h_ref, wq_ref, bq_ref, wk_ref, bk_ref, wv1_ref, bv1_ref,
               wv2_ref, bv2_ref, q_ref, k_ref, v_ref):
    hb = h_ref[...]
    q_ref[...] = jnp.dot(hb, wq_ref[...], preferred_element_type=f32) + bq_ref[...]
    k_ref[...] = jnp.dot(hb, wk_ref[...], preferred_element_type=f32) + bk_ref[...]
    t = jnp.dot(hb, wv1_ref[...], preferred_element_type=f32) + bv1_ref[...]
    t = t * jax.nn.sigmoid(t)
    v_ref[...] = jnp.dot(t, wv2_ref[...], preferred_element_type=f32) + bv2_ref[...]


def _node_proj(h, W_q, b_q, W_k, b_k, W_v1, b_v1, W_v2, b_v2):
    bn = 2000
    grid = N // bn
    blk = pl.BlockSpec((bn, D), lambda i: (i, 0))
    wblk = pl.BlockSpec((D, D), lambda i: (0, 0))
    bblk = pl.BlockSpec((1, D), lambda i: (0, 0))
    out = jax.ShapeDtypeStruct((N, D), f32)
    return pl.pallas_call(
        _proj_body,
        grid=(grid,),
        in_specs=[blk, wblk, bblk, wblk, bblk, wblk, bblk, wblk, bblk],
        out_specs=[blk, blk, blk],
        out_shape=[out, out, out],
    )(h, W_q, b_q.reshape(1, D), W_k, b_k.reshape(1, D),
      W_v1, b_v1.reshape(1, D), W_v2, b_v2.reshape(1, D))


def _re_body(t_ref, w_ref, b_ref, o_ref):
    t = jnp.dot(t_ref[...], w_ref[...], preferred_element_type=f32) + b_ref[...]
    o_ref[...] = t * jax.nn.sigmoid(t)


def _re_proj(t_ij, W_re, b_re):
    be = 4000
    grid = E // be
    return pl.pallas_call(
        _re_body,
        grid=(grid,),
        in_specs=[pl.BlockSpec((be, ED), lambda i: (i, 0)),
                  pl.BlockSpec((ED, D), lambda i: (0, 0)),
                  pl.BlockSpec((1, D), lambda i: (0, 0))],
        out_specs=pl.BlockSpec((be, D), lambda i: (i, 0)),
        out_shape=jax.ShapeDtypeStruct((E, D), f32),
    )(t_ij, W_re, b_re.reshape(1, D))


def _maxmerge_body(mp_ref, o_ref):
    m = jnp.max(mp_ref[...], axis=0)
    o_ref[...] = jnp.concatenate([m, jnp.zeros_like(m)], axis=1)


def _max_merge(mpart):
    bn = 2000
    grid = N // bn
    return pl.pallas_call(
        _maxmerge_body,
        grid=(grid,),
        in_specs=[pl.BlockSpec((NW, bn, H), lambda i: (0, i, 0))],
        out_specs=pl.BlockSpec((bn, 2 * H), lambda i: (i, 0)),
        out_shape=jax.ShapeDtypeStruct((N, 2 * H), f32),
    )(mpart)


def _mr_body(m_ref, s_ref, o_ref):
    m = m_ref[...]
    s = s_ref[0] + s_ref[1]
    r = NORM / (s + 1e-16)
    o_ref[...] = jnp.concatenate([m[:, :H], r[:, :H]], axis=1)


def _mr_merge(m16, spart):
    bn = 2000
    grid = N // bn
    return pl.pallas_call(
        _mr_body,
        grid=(grid,),
        in_specs=[pl.BlockSpec((bn, 2 * H), lambda i: (i, 0)),
                  pl.BlockSpec((NC, bn, 2 * H), lambda i: (0, i, 0))],
        out_specs=pl.BlockSpec((bn, 2 * H), lambda i: (i, 0)),
        out_shape=jax.ShapeDtypeStruct((N, 2 * H), f32),
    )(m16, spart)


def _comb_body(w_ref, wc_ref, bc_ref, o_ref):
    o_ref[...] = (jnp.dot(w_ref[...], wc_ref[...], preferred_element_type=f32)
                  + bc_ref[...])


def _comb_proj(w, W_comb, b_comb):
    be = 4000
    grid = E // be
    return pl.pallas_call(
        _comb_body,
        grid=(grid,),
        in_specs=[pl.BlockSpec((be, D), lambda i: (i, 0)),
                  pl.BlockSpec((D, D), lambda i: (0, 0)),
                  pl.BlockSpec((1, D), lambda i: (0, 0))],
        out_specs=pl.BlockSpec((be, D), lambda i: (i, 0)),
        out_shape=jax.ShapeDtypeStruct((E, D), f32),
    )(w, W_comb, b_comb.reshape(1, D))


def _sc_mesh():
    return plsc.VectorSubcoreMesh(core_axis_name="c", subcore_axis_name="s",
                                  num_cores=NC, num_subcores=NS)


def _wid():
    return lax.axis_index("s") * NC + lax.axis_index("c")


C2 = 80
NCH2 = EPW // C2


def _k2_body(q_hbm, k_hbm, re_hbm, ni_hbm, nj_hbm, neg_hbm,
             l_hbm, mpart_hbm,
             ni_v, nj_v, q_v, k_v, re_v, l_v, acc_v, sem):
    wid = _wid()
    base0 = wid * EPW
    pltpu.sync_copy(neg_hbm, acc_v)
    iota = lax.iota(i32, 16)

    def chunk(ci, carry):
        base = base0 + ci * C2
        pltpu.sync_copy(ni_hbm.at[pl.ds(base, C2)], ni_v)
        pltpu.sync_copy(nj_hbm.at[pl.ds(base, C2)], nj_v)
        pltpu.async_copy(q_hbm.at[ni_v], q_v, sem).wait()
        pltpu.async_copy(k_hbm.at[nj_v], k_v, sem).wait()
        pltpu.sync_copy(re_hbm.at[pl.ds(base, C2)], re_v)

        def edge(e, c2):
            for h in range(H):
                qv = q_v[e, pl.ds(h * DH, DH)]
                kv = k_v[e, pl.ds(h * DH, DH)]
                rv = re_v[e, pl.ds(h * DH, DH)]
                l_v[e, h] = jnp.sum(qv * kv * rv)
            return c2
        lax.fori_loop(0, C2, edge, 0)
        pltpu.sync_copy(l_v, l_hbm.at[pl.ds(base, C2)])

        def grp(g, c2):
            ni16 = ni_v[pl.ds(g * 16, 16)]
            rows = g * 16 + iota
            for h in range(H):
                idx = ni16 * H + h
                val = plsc.load_gather(l_v, [rows, jnp.full((16,), h, i32)])
                cur = plsc.load_gather(acc_v, [idx])
                m0 = val > cur

                def cond(m):
                    return jnp.any(m)

                def body(m):
                    plsc.store_scatter(acc_v, [idx], val, mask=m)
                    cur2 = plsc.load_gather(acc_v, [idx])
                    return val > cur2
                lax.while_loop(cond, body, m0)
            return c2
        lax.fori_loop(0, C2 // 16, grp, 0)
        return carry
    lax.fori_loop(0, NCH2, chunk, 0)
    pltpu.sync_copy(acc_v, mpart_hbm.at[wid])


def _k2_call(q, k, re, ni, nj, neg):
    return pl.kernel(
        _k2_body,
        mesh=_sc_mesh(),
        out_type=(jax.ShapeDtypeStruct((E, H), f32),
                  jax.ShapeDtypeStruct((NW, N * H), f32)),
        scratch_types=[
            pltpu.VMEM((C2,), i32),
            pltpu.VMEM((C2,), i32),
            pltpu.VMEM((C2, D), f32),
            pltpu.VMEM((C2, D), f32),
            pltpu.VMEM((C2, D), f32),
            pltpu.VMEM((C2, H), f32),
            pltpu.VMEM((N * H,), f32),
            pltpu.SemaphoreType.DMA,
        ],
    )(q, k, re, ni, nj, neg)


C5 = 500
NCH5 = EPW // C5
NPS = N // NS


def _k5_body(m16_hbm, l_hbm, ni_hbm, zero_hbm,
             spart_hbm,
             ni_v, m_v, l_v, e_v, s_sp, sem):
    cid = lax.axis_index("c")
    sid = lax.axis_index("s")
    wid = sid * NC + cid
    base0 = wid * EPW
    iota = lax.iota(i32, 16)
    cols = jnp.bitwise_and(iota, 7)

    pltpu.sync_copy(zero_hbm.at[pl.ds(sid * NPS, NPS)],
                    s_sp.at[pl.ds(sid * NPS, NPS)])
    plsc.subcore_barrier()

    def chunk(ci, carry):
        base = base0 + ci * C5
        pltpu.sync_copy(ni_hbm.at[pl.ds(base, C5)], ni_v)
        pltpu.async_copy(m16_hbm.at[ni_v], m_v, sem).wait()
        pltpu.sync_copy(l_hbm.at[pl.ds(base, C5)], l_v)

        def edge(e, c2):
            ldup = plsc.load_gather(l_v, [jnp.full((16,), e, i32), cols])
            m16 = m_v[e, :]
            ev = jnp.exp(ldup - m16)
            e_v[e, :] = jnp.where(iota < 8, ev, 0.0)
            return c2
        lax.fori_loop(0, C5, edge, 0)
        pltpu.sync_copy(e_v, s_sp.at[ni_v], add=True)
        return carry
    lax.fori_loop(0, NCH5, chunk, 0)

    plsc.subcore_barrier()
    pltpu.sync_copy(s_sp.at[pl.ds(sid * NPS, NPS)],
                    spart_hbm.at[cid, pl.ds(sid * NPS, NPS)])


def _k5_call(m16, l, ni, zero):
    return pl.kernel(
        _k5_body,
        mesh=_sc_mesh(),
        out_type=jax.ShapeDtypeStruct((NC, N, 2 * H), f32),
        scratch_types=[
            pltpu.VMEM((C5,), i32),
            pltpu.VMEM((C5, 2 * H), f32),
            pltpu.VMEM((C5, H), f32),
            pltpu.VMEM((C5, 2 * H), f32),
            pltpu.VMEM_SHARED((N, 2 * H), f32),
            pltpu.SemaphoreType.DMA,
        ],
    )(m16, l, ni, zero)


C7 = 200
NCH7 = EPW // C7


def _k7_body(mr_hbm, v_hbm, l_hbm, ni_hbm, nj_hbm,
             w_hbm,
             ni_v, nj_v, mr_v, l_v, v_v, w_v, sem):
    wid = _wid()
    base0 = wid * EPW
    iota = lax.iota(i32, 16)
    cols = jnp.bitwise_and(iota, 7)
    colr = cols + 8

    def chunk(ci, carry):
        base = base0 + ci * C7
        pltpu.sync_copy(ni_hbm.at[pl.ds(base, C7)], ni_v)
        pltpu.sync_copy(nj_hbm.at[pl.ds(base, C7)], nj_v)
        pltpu.async_copy(mr_hbm.at[ni_v], mr_v, sem).wait()
        pltpu.async_copy(v_hbm.at[nj_v], v_v, sem).wait()
        pltpu.sync_copy(l_hbm.at[pl.ds(base, C7)], l_v)

        def edge(e, c2):
            erow = jnp.full((16,), e, i32)
            ldup = plsc.load_gather(l_v, [erow, cols])
            mdup = plsc.load_gather(mr_v, [erow, cols])
            rdup = plsc.load_gather(mr_v, [erow, colr])
            ad = jnp.exp(ldup - mdup) * rdup
            for h in range(H):
                a_s = ad[h]
                w_v[e, pl.ds(h * DH, DH)] = v_v[e, pl.ds(h * DH, DH)] * a_s
            return c2
        lax.fori_loop(0, C7, edge, 0)
        pltpu.sync_copy(w_v, w_hbm.at[pl.ds(base, C7)])
        return carry
    lax.fori_loop(0, NCH7, chunk, 0)


def _k7_call(mr, v, l, ni, nj):
    return pl.kernel(
        _k7_body,
        mesh=_sc_mesh(),
        out_type=jax.ShapeDtypeStruct((E, D), f32),
        scratch_types=[
            pltpu.VMEM((C7,), i32),
            pltpu.VMEM((C7,), i32),
            pltpu.VMEM((C7, 2 * H), f32),
            pltpu.VMEM((C7, H), f32),
            pltpu.VMEM((C7, D), f32),
            pltpu.VMEM((C7, D), f32),
            pltpu.SemaphoreType.DMA,
        ],
    )(mr, v, l, ni, nj)


def kernel(h, t_ij, edge_index, W_q, b_q, W_k, b_k, W_v1, b_v1, W_v2, b_v2,
           W_re, b_re, W_comb, b_comb):
    n_j = edge_index[0].astype(i32)
    n_i = edge_index[1].astype(i32)

    q, k, v = _node_proj(h, W_q, b_q, W_k, b_k, W_v1, b_v1, W_v2, b_v2)
    re = _re_proj(t_ij, W_re, b_re)

    neg = jnp.full((N * H,), NEG, f32)
    l, mpart = _k2_call(q, k, re, n_i, n_j, neg)

    m16 = _max_merge(mpart.reshape(NW, N, H))

    zero = jnp.zeros((N, 2 * H), f32)
    spart = _k5_call(m16, l, n_i, zero)

    mr = _mr_merge(m16, spart)

    w = _k7_call(mr, v, l, n_i, n_j)

    return _comb_proj(w, W_comb, b_comb)


# speedup vs baseline: 3.7406x; 3.7406x over previous
import functools
import math

import jax
import jax.numpy as jnp
from jax import lax
from jax.experimental import pallas as pl
from jax.experimental.pallas import tpu as pltpu
from jax.experimental.pallas import tpu_sc as plsc

N = 10000
E = 320000
D = 128
H = 8
DH = 16
ED = 16
NORM = 1.0 / math.sqrt(DH)

NC = 2
NS = 16
NW = NC * NS
EPW = E // NW
N2 = 10240
NEG = -3.4028235e38

f32 = jnp.float32
i32 = jnp.int32


def _proj_body(h_ref, wq_ref, bq_ref, wk_ref, bk_ref, wv1_ref, bv1_ref,
               wv2_ref, bv2_ref, q_ref, k_ref, v_ref):
    hb = h_ref[...]
    q_ref[...] = jnp.dot(hb, wq_ref[...], preferred_element_type=f32) + bq_ref[...]
    k_ref[...] = jnp.dot(hb, wk_ref[...], preferred_element_type=f32) + bk_ref[...]
    t = jnp.dot(hb, wv1_ref[...], preferred_element_type=f32) + bv1_ref[...]
    t = t * jax.nn.sigmoid(t)
    v_ref[...] = jnp.dot(t, wv2_ref[...], preferred_element_type=f32) + bv2_ref[...]


def _node_proj(h, W_q, b_q, W_k, b_k, W_v1, b_v1, W_v2, b_v2):
    bn = 2000
    grid = N // bn
    blk = pl.BlockSpec((bn, D), lambda i: (i, 0))
    wblk = pl.BlockSpec((D, D), lambda i: (0, 0))
    bblk = pl.BlockSpec((1, D), lambda i: (0, 0))
    out = jax.ShapeDtypeStruct((N, D), f32)
    return pl.pallas_call(
        _proj_body,
        grid=(grid,),
        in_specs=[blk, wblk, bblk, wblk, bblk, wblk, bblk, wblk, bblk],
        out_specs=[blk, blk, blk],
        out_shape=[out, out, out],
    )(h, W_q, b_q.reshape(1, D), W_k, b_k.reshape(1, D),
      W_v1, b_v1.reshape(1, D), W_v2, b_v2.reshape(1, D))


def _re_body(t_ref, w_ref, b_ref, o_ref):
    t = jnp.dot(t_ref[...], w_ref[...], preferred_element_type=f32) + b_ref[...]
    o_ref[...] = t * jax.nn.sigmoid(t)


def _re_proj(t_ij, W_re, b_re):
    be = 4000
    grid = E // be
    return pl.pallas_call(
        _re_body,
        grid=(grid,),
        in_specs=[pl.BlockSpec((be, ED), lambda i: (i, 0)),
                  pl.BlockSpec((ED, D), lambda i: (0, 0)),
                  pl.BlockSpec((1, D), lambda i: (0, 0))],
        out_specs=pl.BlockSpec((be, D), lambda i: (i, 0)),
        out_shape=jax.ShapeDtypeStruct((E, D), f32),
    )(t_ij, W_re, b_re.reshape(1, D))


def _smerge_body(mp_ref, sp_ref, m_ref, r_ref):
    mp = mp_ref[...]
    sp = sp_ref[...]
    m = jnp.max(mp, axis=0)
    s = jnp.sum(sp * jnp.exp(mp - m[None]), axis=0)
    m_ref[...] = m
    r_ref[...] = NORM / (s + 1e-16)


def _softmax_merge(mpart, spart):
    rows = N2 * H // D
    bn = 160
    grid = rows // bn
    blk3 = pl.BlockSpec((NW, bn, D), lambda i: (0, i, 0))
    blk2 = pl.BlockSpec((bn, D), lambda i: (i, 0))
    out = jax.ShapeDtypeStruct((rows, D), f32)
    return pl.pallas_call(
        _smerge_body,
        grid=(grid,),
        in_specs=[blk3, blk3],
        out_specs=[blk2, blk2],
        out_shape=[out, out],
    )(mpart, spart)


def _comb_body(w_ref, wc_ref, bc_ref, o_ref):
    o_ref[...] = (jnp.dot(w_ref[...], wc_ref[...], preferred_element_type=f32)
                  + bc_ref[...])


def _comb_proj(w, W_comb, b_comb):
    be = 4000
    grid = E // be
    return pl.pallas_call(
        _comb_body,
        grid=(grid,),
        in_specs=[pl.BlockSpec((be, D), lambda i: (i, 0)),
                  pl.BlockSpec((D, D), lambda i: (0, 0)),
                  pl.BlockSpec((1, D), lambda i: (0, 0))],
        out_specs=pl.BlockSpec((be, D), lambda i: (i, 0)),
        out_shape=jax.ShapeDtypeStruct((E, D), f32),
    )(w, W_comb, b_comb.reshape(1, D))


def _sc_mesh():
    return plsc.VectorSubcoreMesh(core_axis_name="c", subcore_axis_name="s",
                                  num_cores=NC, num_subcores=NS)


def _wid():
    return lax.axis_index("s") * NC + lax.axis_index("c")


def _al8(x):
    return pl.multiple_of(x, 8)


_SC_PARAMS = pltpu.CompilerParams(needs_layout_passes=False)

B2 = 400
G2 = 80
NSUB = B2 // G2
NBLK2 = EPW // B2


def _k2a_body(q_hbm, k_hbm, re_hbm, ni_hbm, nj_hbm,
              l_hbm,
              ni_v, nj_v, q_v, k_v, re_v, l_v, sem):
    wid = _wid()
    base0 = wid * EPW
    iota = lax.iota(i32, 16)

    def block(bi, carry):
        base = _al8(base0 + bi * B2)
        pltpu.sync_copy(ni_hbm.at[pl.ds(base, B2)], ni_v)
        pltpu.sync_copy(nj_hbm.at[pl.ds(base, B2)], nj_v)
        for sub in range(NSUB):
            pltpu.async_copy(q_hbm.at[ni_v.at[pl.ds(sub * G2, G2)]],
                             q_v, sem).wait()
            pltpu.async_copy(k_hbm.at[nj_v.at[pl.ds(sub * G2, G2)]],
                             k_v, sem).wait()
            pltpu.sync_copy(re_hbm.at[pl.ds(base + sub * G2, G2)], re_v)

            def edge(ep, c2):
                lacc = jnp.zeros((16,), f32)
                for half in range(2):
                    e = 2 * ep + half
                    for h in range(H):
                        qv = q_v[e, pl.ds(h * DH, DH)]
                        kv = k_v[e, pl.ds(h * DH, DH)]
                        rv = re_v[e, pl.ds(h * DH, DH)]
                        s = jnp.sum(qv * kv * rv)
                        lacc = jnp.where(iota == (half * H + h),
                                         jnp.broadcast_to(s, (16,)), lacc)
                l_v[pl.ds(_al8((sub * (G2 // 2) + ep) * 16), 16)] = lacc
                return c2
            lax.fori_loop(0, G2 // 2, edge, 0)
        pltpu.sync_copy(l_v, l_hbm.at[pl.ds(_al8(base * H), B2 * H)])
        return carry
    lax.fori_loop(0, NBLK2, block, 0)


def _k2a_call(q, k, re, ni, nj):
    return pl.kernel(
        _k2a_body,
        mesh=_sc_mesh(),
        compiler_params=_SC_PARAMS,
        out_type=jax.ShapeDtypeStruct((E * H,), f32),
        scratch_types=[
            pltpu.VMEM((B2,), i32),
            pltpu.VMEM((B2,), i32),
            pltpu.VMEM((G2, D), f32),
            pltpu.VMEM((G2, D), f32),
            pltpu.VMEM((G2, D), f32),
            pltpu.VMEM((B2 * H,), f32),
            pltpu.SemaphoreType.DMA,
        ],
    )(q, k, re, ni, nj)


B2B = 2000
NBLK2B = EPW // B2B


def _k2b_body(l_hbm, ni_hbm, neg_hbm,
              mpart_hbm,
              ni_v, l_v, acc_v, sem):
    wid = _wid()
    base0 = wid * EPW
    pltpu.sync_copy(neg_hbm, acc_v)
    iota = lax.iota(i32, 16)

    def block(bi, carry):
        base = _al8(base0 + bi * B2B)
        pltpu.sync_copy(ni_hbm.at[pl.ds(base, B2B)], ni_v)
        pltpu.sync_copy(l_hbm.at[pl.ds(_al8(base * H), B2B * H)], l_v)

        def grp(g, c2):
            ni16 = ni_v[pl.ds(g * 16, 16)]
            el = g * 16 + iota
            for h in range(H):
                f = ni16 * H + h
                arow = lax.shift_right_logical(f, 7)
                acol = jnp.bitwise_and(f, 127)
                val = plsc.load_gather(l_v, [el * H + h])
                cur = plsc.load_gather(acc_v, [arow, acol])
                m0 = val > cur

                def cond(m):
                    return jnp.any(m)

                def body(m):
                    plsc.store_scatter(acc_v, [arow, acol], val, mask=m)
                    cur2 = plsc.load_gather(acc_v, [arow, acol])
                    return val > cur2
                lax.while_loop(cond, body, m0)
            return c2
        lax.fori_loop(0, B2B // 16, grp, 0)
        return carry
    lax.fori_loop(0, NBLK2B, block, 0)
    pltpu.sync_copy(acc_v, mpart_hbm.at[wid])


def _k2b_call(l, ni, neg):
    return pl.kernel(
        _k2b_body,
        mesh=_sc_mesh(),
        compiler_params=_SC_PARAMS,
        out_type=jax.ShapeDtypeStruct((NW, N2 * H // D, D), f32),
        scratch_types=[
            pltpu.VMEM((B2B,), i32),
            pltpu.VMEM((B2B * H,), f32),
            pltpu.VMEM((N2 * H // D, D), f32),
            pltpu.SemaphoreType.DMA,
        ],
    )(l, ni, neg)


B5 = 400
NBLK5 = EPW // B5
NHALF = N2 * H // D // 2


def _k5_body(half, l_hbm, ni_hbm, mpart_hbm, zero_hbm,
             spart_hbm,
             ni_v, l_v, m_acc, s_acc, sem):
    wid = _wid()
    base0 = wid * EPW
    iota = lax.iota(i32, 16)
    fbase = half * (NHALF * D)

    pltpu.sync_copy(mpart_hbm.at[wid, pl.ds(half * NHALF, NHALF)], m_acc)
    pltpu.sync_copy(zero_hbm, s_acc)

    def block(bi, carry):
        base = _al8(base0 + bi * B5)
        pltpu.sync_copy(ni_hbm.at[pl.ds(base, B5)], ni_v)
        pltpu.sync_copy(l_hbm.at[pl.ds(_al8(base * H), B5 * H)], l_v)

        def grp(g, c2):
            ni16 = ni_v[pl.ds(g * 16, 16)]
            el = g * 16 + iota
            for h in range(H):
                f = ni16 * H + h - fbase
                valid = jnp.logical_and(f >= 0, f < NHALF * D)
                fc = jnp.clip(f, 0, NHALF * D - 1)
                arow = lax.shift_right_logical(fc, 7)
                acol = jnp.bitwise_and(fc, 127)
                lv = plsc.load_gather(l_v, [el * H + h])
                mv = plsc.load_gather(m_acc, [arow, acol])
                ev = jnp.exp(lv - mv)
                plsc.addupdate_scatter(s_acc, [arow, acol], ev, mask=valid)
            return c2
        lax.fori_loop(0, B5 // 16, grp, 0)
        return carry
    lax.fori_loop(0, NBLK5, block, 0)
    pltpu.sync_copy(s_acc, spart_hbm.at[wid])


def _k5_call(half, l, ni, mpart, zero):
    return pl.kernel(
        functools.partial(_k5_body, half),
        mesh=_sc_mesh(),
        compiler_params=_SC_PARAMS,
        out_type=jax.ShapeDtypeStruct((NW, NHALF, D), f32),
        scratch_types=[
            pltpu.VMEM((B5,), i32),
            pltpu.VMEM((B5 * H,), f32),
            pltpu.VMEM((NHALF, D), f32),
            pltpu.VMEM((NHALF, D), f32),
            pltpu.SemaphoreType.DMA,
        ],
    )(l, ni, mpart, zero)


C7 = 80
NCH7 = EPW // C7


def _k7_body(mr_hbm, v_hbm, l_hbm, ni_hbm, nj_hbm,
             w_hbm,
             ni_v, nj_v, mr_v, l_v, v_v, w_v, sem):
    wid = _wid()
    base0 = wid * EPW
    iota = lax.iota(i32, 16)
    cols = jnp.bitwise_and(iota, 7)
    colr = cols + 8

    def chunk(ci, carry):
        base = _al8(base0 + ci * C7)
        pltpu.sync_copy(ni_hbm.at[pl.ds(base, C7)], ni_v)
        pltpu.sync_copy(nj_hbm.at[pl.ds(base, C7)], nj_v)
        pltpu.async_copy(mr_hbm.at[ni_v], mr_v, sem).wait()
        pltpu.async_copy(v_hbm.at[nj_v], v_v, sem).wait()
        pltpu.sync_copy(l_hbm.at[pl.ds(_al8(base * H), C7 * H)], l_v)

        def edge(e, c2):
            erow = jnp.broadcast_to(e, (16,))
            lidx = jnp.broadcast_to(e * H, (16,)) + cols
            ldup = plsc.load_gather(l_v, [lidx])
            mdup = plsc.load_gather(mr_v, [erow, cols])
            rdup = plsc.load_gather(mr_v, [erow, colr])
            ad = jnp.exp(ldup - mdup) * rdup
            for h in range(H):
                a_s = ad[h]
                w_v[e, pl.ds(h * DH, DH)] = v_v[e, pl.ds(h * DH, DH)] * a_s
            return c2
        lax.fori_loop(0, C7, edge, 0)
        pltpu.sync_copy(w_v, w_hbm.at[pl.ds(base, C7)])
        return carry
    lax.fori_loop(0, NCH7, chunk, 0)


def _k7_call(mr, v, l, ni, nj):
    return pl.kernel(
        _k7_body,
        mesh=_sc_mesh(),
        compiler_params=_SC_PARAMS,
        out_type=jax.ShapeDtypeStruct((E, D), f32),
        scratch_types=[
            pltpu.VMEM((C7,), i32),
            pltpu.VMEM((C7,), i32),
            pltpu.VMEM((C7, D), f32),
            pltpu.VMEM((C7 * H,), f32),
            pltpu.VMEM((C7, D), f32),
            pltpu.VMEM((C7, D), f32),
            pltpu.SemaphoreType.DMA,
        ],
    )(mr, v, l, ni, nj)


def kernel(h, t_ij, edge_index, W_q, b_q, W_k, b_k, W_v1, b_v1, W_v2, b_v2,
           W_re, b_re, W_comb, b_comb):
    n_j = edge_index[0].astype(i32)
    n_i = edge_index[1].astype(i32)

    q, k, v = _node_proj(h, W_q, b_q, W_k, b_k, W_v1, b_v1, W_v2, b_v2)
    re = _re_proj(t_ij, W_re, b_re)

    l = _k2a_call(q, k, re, n_i, n_j)

    neg = jnp.full((N2 * H // D, D), NEG, f32)
    mpart = _k2b_call(l, n_i, neg)

    zero = jnp.zeros((NHALF, D), f32)
    spart0 = _k5_call(0, l, n_i, mpart, zero)
    spart1 = _k5_call(1, l, n_i, mpart, zero)

    spart = jnp.concatenate([spart0, spart1], axis=1)
    mflat, rflat = _softmax_merge(mpart, spart)
    mr = jnp.concatenate([mflat.reshape(N2, H), rflat.reshape(N2, H),
                          jnp.zeros((N2, D - 2 * H), f32)], axis=1)

    w = _k7_call(mr, v, l, n_i, n_j)

    return _comb_proj(w, W_comb, b_comb)


# speedup vs baseline: 5.7212x; 1.5295x over previous
import functools
import math

import jax
import jax.numpy as jnp
from jax import lax
from jax.experimental import pallas as pl
from jax.experimental.pallas import tpu as pltpu
from jax.experimental.pallas import tpu_sc as plsc

N = 10000
E = 320000
D = 128
H = 8
DH = 16
ED = 16
NORM = 1.0 / math.sqrt(DH)

NC = 2
NS = 16
NW = NC * NS
EPW = E // NW
N2 = 10240
NEG = -3.4028235e38

f32 = jnp.float32
i32 = jnp.int32


def _proj_body(h_ref, wq_ref, bq_ref, wk_ref, bk_ref, wv1_ref, bv1_ref,
               wv2_ref, bv2_ref, q_ref, k_ref, v_ref):
    hb = h_ref[...]
    q_ref[...] = jnp.dot(hb, wq_ref[...], preferred_element_type=f32) + bq_ref[...]
    k_ref[...] = jnp.dot(hb, wk_ref[...], preferred_element_type=f32) + bk_ref[...]
    t = jnp.dot(hb, wv1_ref[...], preferred_element_type=f32) + bv1_ref[...]
    t = t * jax.nn.sigmoid(t)
    v_ref[...] = jnp.dot(t, wv2_ref[...], preferred_element_type=f32) + bv2_ref[...]


def _node_proj(h, W_q, b_q, W_k, b_k, W_v1, b_v1, W_v2, b_v2):
    bn = 2000
    grid = N // bn
    blk = pl.BlockSpec((bn, D), lambda i: (i, 0))
    wblk = pl.BlockSpec((D, D), lambda i: (0, 0))
    bblk = pl.BlockSpec((1, D), lambda i: (0, 0))
    out = jax.ShapeDtypeStruct((N, D), f32)
    return pl.pallas_call(
        _proj_body,
        grid=(grid,),
        in_specs=[blk, wblk, bblk, wblk, bblk, wblk, bblk, wblk, bblk],
        out_specs=[blk, blk, blk],
        out_shape=[out, out, out],
    )(h, W_q, b_q.reshape(1, D), W_k, b_k.reshape(1, D),
      W_v1, b_v1.reshape(1, D), W_v2, b_v2.reshape(1, D))


def _re_body(t_ref, w_ref, b_ref, o_ref):
    t = jnp.dot(t_ref[...], w_ref[...], preferred_element_type=f32) + b_ref[...]
    o_ref[...] = t * jax.nn.sigmoid(t)


def _re_proj(t_ij, W_re, b_re):
    be = 4000
    grid = E // be
    return pl.pallas_call(
        _re_body,
        grid=(grid,),
        in_specs=[pl.BlockSpec((be, ED), lambda i: (i, 0)),
                  pl.BlockSpec((ED, D), lambda i: (0, 0)),
                  pl.BlockSpec((1, D), lambda i: (0, 0))],
        out_specs=pl.BlockSpec((be, D), lambda i: (i, 0)),
        out_shape=jax.ShapeDtypeStruct((E, D), f32),
    )(t_ij, W_re, b_re.reshape(1, D))


def _smerge_body(mp_ref, sp_ref, m_ref, r_ref):
    mp = mp_ref[...]
    sp = sp_ref[...]
    m = jnp.max(mp, axis=0)
    s = jnp.sum(sp * jnp.exp(mp - m[None]), axis=0)
    m_ref[...] = m
    r_ref[...] = NORM / (s + 1e-16)


def _softmax_merge(mpart, spart):
    rows = N2 * H // D
    bn = 160
    grid = rows // bn
    blk3 = pl.BlockSpec((NW, bn, D), lambda i: (0, i, 0))
    blk2 = pl.BlockSpec((bn, D), lambda i: (i, 0))
    out = jax.ShapeDtypeStruct((rows, D), f32)
    return pl.pallas_call(
        _smerge_body,
        grid=(grid,),
        in_specs=[blk3, blk3],
        out_specs=[blk2, blk2],
        out_shape=[out, out],
    )(mpart, spart)


def _comb_body(w_ref, wc_ref, bc_ref, o_ref):
    o_ref[...] = (jnp.dot(w_ref[...], wc_ref[...], preferred_element_type=f32)
                  + bc_ref[...])


def _comb_proj(w, W_comb, b_comb):
    be = 4000
    grid = E // be
    return pl.pallas_call(
        _comb_body,
        grid=(grid,),
        in_specs=[pl.BlockSpec((be, D), lambda i: (i, 0)),
                  pl.BlockSpec((D, D), lambda i: (0, 0)),
                  pl.BlockSpec((1, D), lambda i: (0, 0))],
        out_specs=pl.BlockSpec((be, D), lambda i: (i, 0)),
        out_shape=jax.ShapeDtypeStruct((E, D), f32),
    )(w, W_comb, b_comb.reshape(1, D))


def _sc_mesh():
    return plsc.VectorSubcoreMesh(core_axis_name="c", subcore_axis_name="s",
                                  num_cores=NC, num_subcores=NS)


def _wid():
    return lax.axis_index("s") * NC + lax.axis_index("c")


def _al8(x):
    return pl.multiple_of(x, 8)


_SC_PARAMS = pltpu.CompilerParams(needs_layout_passes=False)

G2 = 80
NCH2 = EPW // G2


def _k2a_body(q_hbm, k_hbm, re_hbm, ni_hbm, nj_hbm,
              l_hbm,
              ni_v, nj_v,
              q0, k0, re0, q1, k1, re1, l0, l1,
              gs0, gs1, ls0, ls1):
    wid = _wid()
    base0 = wid * EPW
    iota = lax.iota(i32, 16)
    pltpu.sync_copy(ni_hbm.at[pl.ds(_al8(base0), EPW)], ni_v)
    pltpu.sync_copy(nj_hbm.at[pl.ds(_al8(base0), EPW)], nj_v)
    bufs = ((q0, k0, re0, l0, gs0, ls0), (q1, k1, re1, l1, gs1, ls1))

    def issue(c, b):
        qb, kb, rb, _, gs, _ = bufs[b]
        off = _al8(c * G2)
        pltpu.async_copy(q_hbm.at[ni_v.at[pl.ds(off, G2)]], qb, gs)
        pltpu.async_copy(k_hbm.at[nj_v.at[pl.ds(off, G2)]], kb, gs)
        pltpu.async_copy(re_hbm.at[pl.ds(_al8(base0 + c * G2), G2)], rb, gs)

    def wait_gather(b):
        qb, kb, rb, _, gs, _ = bufs[b]
        pltpu.make_async_copy(q_hbm.at[ni_v.at[pl.ds(0, G2)]], qb, gs).wait()
        pltpu.make_async_copy(k_hbm.at[nj_v.at[pl.ds(0, G2)]], kb, gs).wait()
        pltpu.make_async_copy(re_hbm.at[pl.ds(0, G2)], rb, gs).wait()

    def wait_lwrite(b):
        _, _, _, lb, _, ls = bufs[b]
        pltpu.make_async_copy(lb, l_hbm.at[pl.ds(0, G2 * H)], ls).wait()

    def compute(c, b):
        qb, kb, rb, lb, _, ls = bufs[b]

        def edge(ep, c2):
            lacc = jnp.zeros((16,), f32)
            for half in range(2):
                e = 2 * ep + half
                for h in range(H):
                    qv = qb[e, pl.ds(h * DH, DH)]
                    kv = kb[e, pl.ds(h * DH, DH)]
                    rv = rb[e, pl.ds(h * DH, DH)]
                    s = jnp.sum(qv * kv * rv)
                    lacc = jnp.where(iota == (half * H + h),
                                     jnp.broadcast_to(s, (16,)), lacc)
            lb[pl.ds(_al8(ep * 16), 16)] = lacc
            return c2
        lax.fori_loop(0, G2 // 2, edge, 0)
        pltpu.async_copy(
            lb, l_hbm.at[pl.ds(_al8((base0 + c * G2) * H), G2 * H)], ls)

    issue(0, 0)
    issue(1, 1)

    def sup(s2, carry):
        for b in range(2):
            c = 2 * s2 + b
            wait_gather(b)

            @pl.when(c >= 2)
            def _():
                wait_lwrite(b)
            compute(c, b)

            @pl.when(c + 2 < NCH2)
            def _():
                issue(c + 2, b)
        return carry
    lax.fori_loop(0, (NCH2 - 1) // 2, sup, 0)
    wait_gather(0)
    wait_lwrite(0)
    compute(NCH2 - 1, 0)
    wait_lwrite(1)
    wait_lwrite(0)


def _k2a_call(q, k, re, ni, nj):
    return pl.kernel(
        _k2a_body,
        mesh=_sc_mesh(),
        compiler_params=_SC_PARAMS,
        out_type=jax.ShapeDtypeStruct((E * H,), f32),
        scratch_types=[
            pltpu.VMEM((EPW,), i32),
            pltpu.VMEM((EPW,), i32),
            pltpu.VMEM((G2, D), f32),
            pltpu.VMEM((G2, D), f32),
            pltpu.VMEM((G2, D), f32),
            pltpu.VMEM((G2, D), f32),
            pltpu.VMEM((G2, D), f32),
            pltpu.VMEM((G2, D), f32),
            pltpu.VMEM((G2 * H,), f32),
            pltpu.VMEM((G2 * H,), f32),
            pltpu.SemaphoreType.DMA,
            pltpu.SemaphoreType.DMA,
            pltpu.SemaphoreType.DMA,
            pltpu.SemaphoreType.DMA,
        ],
    )(q, k, re, ni, nj)


B2B = 2000
NBLK2B = EPW // B2B


def _k2b_body(l_hbm, ni_hbm, neg_hbm,
              mpart_hbm,
              ni_v, l_v, acc_v, sem):
    wid = _wid()
    base0 = wid * EPW
    pltpu.sync_copy(neg_hbm, acc_v)
    iota = lax.iota(i32, 16)

    def block(bi, carry):
        base = _al8(base0 + bi * B2B)
        pltpu.sync_copy(ni_hbm.at[pl.ds(base, B2B)], ni_v)
        pltpu.sync_copy(l_hbm.at[pl.ds(_al8(base * H), B2B * H)], l_v)

        def grp(g, c2):
            ni16 = ni_v[pl.ds(g * 16, 16)]
            el = g * 16 + iota
            for h in range(H):
                f = ni16 * H + h
                arow = lax.shift_right_logical(f, 7)
                acol = jnp.bitwise_and(f, 127)
                val = plsc.load_gather(l_v, [el * H + h])
                cur = plsc.load_gather(acc_v, [arow, acol])
                m0 = val > cur

                def cond(m):
                    return jnp.any(m)

                def body(m):
                    plsc.store_scatter(acc_v, [arow, acol], val, mask=m)
                    cur2 = plsc.load_gather(acc_v, [arow, acol])
                    return val > cur2
                lax.while_loop(cond, body, m0)
            return c2
        lax.fori_loop(0, B2B // 16, grp, 0)
        return carry
    lax.fori_loop(0, NBLK2B, block, 0)
    pltpu.sync_copy(acc_v, mpart_hbm.at[wid])


def _k2b_call(l, ni, neg):
    return pl.kernel(
        _k2b_body,
        mesh=_sc_mesh(),
        compiler_params=_SC_PARAMS,
        out_type=jax.ShapeDtypeStruct((NW, N2 * H // D, D), f32),
        scratch_types=[
            pltpu.VMEM((B2B,), i32),
            pltpu.VMEM((B2B * H,), f32),
            pltpu.VMEM((N2 * H // D, D), f32),
            pltpu.SemaphoreType.DMA,
        ],
    )(l, ni, neg)


B5 = 400
NBLK5 = EPW // B5
NHALF = N2 * H // D // 2


def _k5_body(half, l_hbm, ni_hbm, mpart_hbm, zero_hbm,
             spart_hbm,
             ni_v, l_v, m_acc, s_acc, sem):
    wid = _wid()
    base0 = wid * EPW
    iota = lax.iota(i32, 16)
    fbase = half * (NHALF * D)

    pltpu.sync_copy(mpart_hbm.at[wid, pl.ds(half * NHALF, NHALF)], m_acc)
    pltpu.sync_copy(zero_hbm, s_acc)

    def block(bi, carry):
        base = _al8(base0 + bi * B5)
        pltpu.sync_copy(ni_hbm.at[pl.ds(base, B5)], ni_v)
        pltpu.sync_copy(l_hbm.at[pl.ds(_al8(base * H), B5 * H)], l_v)

        def grp(g, c2):
            ni16 = ni_v[pl.ds(g * 16, 16)]
            el = g * 16 + iota
            for h in range(H):
                f = ni16 * H + h - fbase
                valid = jnp.logical_and(f >= 0, f < NHALF * D)
                fc = jnp.clip(f, 0, NHALF * D - 1)
                arow = lax.shift_right_logical(fc, 7)
                acol = jnp.bitwise_and(fc, 127)
                lv = plsc.load_gather(l_v, [el * H + h])
                mv = plsc.load_gather(m_acc, [arow, acol])
                ev = jnp.exp(lv - mv)
                plsc.addupdate_scatter(s_acc, [arow, acol], ev, mask=valid)
            return c2
        lax.fori_loop(0, B5 // 16, grp, 0)
        return carry
    lax.fori_loop(0, NBLK5, block, 0)
    pltpu.sync_copy(s_acc, spart_hbm.at[wid])


def _k5_call(half, l, ni, mpart, zero):
    return pl.kernel(
        functools.partial(_k5_body, half),
        mesh=_sc_mesh(),
        compiler_params=_SC_PARAMS,
        out_type=jax.ShapeDtypeStruct((NW, NHALF, D), f32),
        scratch_types=[
            pltpu.VMEM((B5,), i32),
            pltpu.VMEM((B5 * H,), f32),
            pltpu.VMEM((NHALF, D), f32),
            pltpu.VMEM((NHALF, D), f32),
            pltpu.SemaphoreType.DMA,
        ],
    )(l, ni, mpart, zero)


G7 = 80
NCH7 = EPW // G7


def _k7_body(mr_hbm, v_hbm, l_hbm, ni_hbm, nj_hbm,
             w_hbm,
             ni_v, nj_v, mr0, v0, l0, mr1, v1, l1, w0, w1,
             gs0, gs1, ws0, ws1):
    wid = _wid()
    base0 = wid * EPW
    iota = lax.iota(i32, 16)
    cols = jnp.bitwise_and(iota, 7)
    colr = cols + 8
    pltpu.sync_copy(ni_hbm.at[pl.ds(_al8(base0), EPW)], ni_v)
    pltpu.sync_copy(nj_hbm.at[pl.ds(_al8(base0), EPW)], nj_v)
    bufs = ((mr0, v0, l0, w0, gs0, ws0), (mr1, v1, l1, w1, gs1, ws1))

    def issue(c, b):
        mrb, vb, lb, _, gs, _ = bufs[b]
        off = _al8(c * G7)
        pltpu.async_copy(mr_hbm.at[ni_v.at[pl.ds(off, G7)]], mrb, gs)
        pltpu.async_copy(v_hbm.at[nj_v.at[pl.ds(off, G7)]], vb, gs)
        pltpu.async_copy(l_hbm.at[pl.ds(_al8((base0 + c * G7) * H), G7 * H)],
                         lb, gs)

    def wait_gather(b):
        mrb, vb, lb, _, gs, _ = bufs[b]
        pltpu.make_async_copy(mr_hbm.at[ni_v.at[pl.ds(0, G7)]], mrb, gs).wait()
        pltpu.make_async_copy(v_hbm.at[nj_v.at[pl.ds(0, G7)]], vb, gs).wait()
        pltpu.make_async_copy(l_hbm.at[pl.ds(0, G7 * H)], lb, gs).wait()

    def wait_wwrite(b):
        _, _, _, wb, _, ws = bufs[b]
        pltpu.make_async_copy(wb, w_hbm.at[pl.ds(0, G7)], ws).wait()

    def compute(c, b):
        mrb, vb, lb, wb, _, ws = bufs[b]

        def edge(e, c2):
            erow = jnp.broadcast_to(e, (16,))
            lidx = jnp.broadcast_to(e * H, (16,)) + cols
            ldup = plsc.load_gather(lb, [lidx])
            mdup = plsc.load_gather(mrb, [erow, cols])
            rdup = plsc.load_gather(mrb, [erow, colr])
            ad = jnp.exp(ldup - mdup) * rdup
            for h in range(H):
                a_s = ad[h]
                wb[e, pl.ds(h * DH, DH)] = vb[e, pl.ds(h * DH, DH)] * a_s
            return c2
        lax.fori_loop(0, G7, edge, 0)
        pltpu.async_copy(wb, w_hbm.at[pl.ds(_al8(base0 + c * G7), G7)], ws)

    issue(0, 0)
    issue(1, 1)

    def sup(s2, carry):
        for b in range(2):
            c = 2 * s2 + b
            wait_gather(b)

            @pl.when(c >= 2)
            def _():
                wait_wwrite(b)
            compute(c, b)

            @pl.when(c + 2 < NCH7)
            def _():
                issue(c + 2, b)
        return carry
    lax.fori_loop(0, (NCH7 - 1) // 2, sup, 0)
    wait_gather(0)
    wait_wwrite(0)
    compute(NCH7 - 1, 0)
    wait_wwrite(1)
    wait_wwrite(0)


def _k7_call(mr, v, l, ni, nj):
    return pl.kernel(
        _k7_body,
        mesh=_sc_mesh(),
        compiler_params=_SC_PARAMS,
        out_type=jax.ShapeDtypeStruct((E, D), f32),
        scratch_types=[
            pltpu.VMEM((EPW,), i32),
            pltpu.VMEM((EPW,), i32),
            pltpu.VMEM((G7, D), f32),
            pltpu.VMEM((G7, D), f32),
            pltpu.VMEM((G7 * H,), f32),
            pltpu.VMEM((G7, D), f32),
            pltpu.VMEM((G7, D), f32),
            pltpu.VMEM((G7 * H,), f32),
            pltpu.VMEM((G7, D), f32),
            pltpu.VMEM((G7, D), f32),
            pltpu.SemaphoreType.DMA,
            pltpu.SemaphoreType.DMA,
            pltpu.SemaphoreType.DMA,
            pltpu.SemaphoreType.DMA,
        ],
    )(mr, v, l, ni, nj)


def kernel(h, t_ij, edge_index, W_q, b_q, W_k, b_k, W_v1, b_v1, W_v2, b_v2,
           W_re, b_re, W_comb, b_comb):
    n_j = edge_index[0].astype(i32)
    n_i = edge_index[1].astype(i32)

    q, k, v = _node_proj(h, W_q, b_q, W_k, b_k, W_v1, b_v1, W_v2, b_v2)
    re = _re_proj(t_ij, W_re, b_re)

    l = _k2a_call(q, k, re, n_i, n_j)

    neg = jnp.full((N2 * H // D, D), NEG, f32)
    mpart = _k2b_call(l, n_i, neg)

    zero = jnp.zeros((NHALF, D), f32)
    spart0 = _k5_call(0, l, n_i, mpart, zero)
    spart1 = _k5_call(1, l, n_i, mpart, zero)

    spart = jnp.concatenate([spart0, spart1], axis=1)
    mflat, rflat = _softmax_merge(mpart, spart)
    mr = jnp.concatenate([mflat.reshape(N2, H), rflat.reshape(N2, H),
                          jnp.zeros((N2, D - 2 * H), f32)], axis=1)

    w = _k7_call(mr, v, l, n_i, n_j)

    return _comb_proj(w, W_comb, b_comb)


# speedup vs baseline: 6.4851x; 1.1335x over previous
import functools
import math

import jax
import jax.numpy as jnp
from jax import lax
from jax.experimental import pallas as pl
from jax.experimental.pallas import tpu as pltpu
from jax.experimental.pallas import tpu_sc as plsc

N = 10000
E = 320000
D = 128
H = 8
DH = 16
ED = 16
NORM = 1.0 / math.sqrt(DH)

NC = 2
NS = 16
NW = NC * NS
EPW = E // NW
N2 = 10240
NEG = -3.4028235e38

f32 = jnp.float32
i32 = jnp.int32


def _proj_body(h_ref, wq_ref, bq_ref, wk_ref, bk_ref, wv1_ref, bv1_ref,
               wv2_ref, bv2_ref, q_ref, k_ref, v_ref):
    hb = h_ref[...]
    q_ref[...] = jnp.dot(hb, wq_ref[...], preferred_element_type=f32) + bq_ref[...]
    k_ref[...] = jnp.dot(hb, wk_ref[...], preferred_element_type=f32) + bk_ref[...]
    t = jnp.dot(hb, wv1_ref[...], preferred_element_type=f32) + bv1_ref[...]
    t = t * jax.nn.sigmoid(t)
    v_ref[...] = jnp.dot(t, wv2_ref[...], preferred_element_type=f32) + bv2_ref[...]


def _node_proj(h, W_q, b_q, W_k, b_k, W_v1, b_v1, W_v2, b_v2):
    bn = 2000
    grid = N // bn
    blk = pl.BlockSpec((bn, D), lambda i: (i, 0))
    wblk = pl.BlockSpec((D, D), lambda i: (0, 0))
    bblk = pl.BlockSpec((1, D), lambda i: (0, 0))
    out = jax.ShapeDtypeStruct((N, D), f32)
    return pl.pallas_call(
        _proj_body,
        grid=(grid,),
        in_specs=[blk, wblk, bblk, wblk, bblk, wblk, bblk, wblk, bblk],
        out_specs=[blk, blk, blk],
        out_shape=[out, out, out],
    )(h, W_q, b_q.reshape(1, D), W_k, b_k.reshape(1, D),
      W_v1, b_v1.reshape(1, D), W_v2, b_v2.reshape(1, D))


def _re_body(t_ref, w_ref, b_ref, o_ref):
    t = jnp.dot(t_ref[...], w_ref[...], preferred_element_type=f32) + b_ref[...]
    o_ref[...] = t * jax.nn.sigmoid(t)


def _re_proj(t_ij, W_re, b_re):
    be = 4000
    grid = E // be
    return pl.pallas_call(
        _re_body,
        grid=(grid,),
        in_specs=[pl.BlockSpec((be, ED), lambda i: (i, 0)),
                  pl.BlockSpec((ED, D), lambda i: (0, 0)),
                  pl.BlockSpec((1, D), lambda i: (0, 0))],
        out_specs=pl.BlockSpec((be, D), lambda i: (i, 0)),
        out_shape=jax.ShapeDtypeStruct((E, D), f32),
    )(t_ij, W_re, b_re.reshape(1, D))


def _smerge_body(mp_ref, sp_ref, g_ref):
    mp = mp_ref[...]
    sp = sp_ref[...]
    m = jnp.max(mp, axis=0)
    s = jnp.sum(sp * jnp.exp(mp - m[None]), axis=0)
    g_ref[...] = m - math.log(NORM) + jnp.log(s + 1e-16)


def _softmax_merge(mpart, spart):
    rows = N2 * H // D
    bn = 160
    grid = rows // bn
    blk3 = pl.BlockSpec((NW, bn, D), lambda i: (0, i, 0))
    blk2 = pl.BlockSpec((bn, D), lambda i: (i, 0))
    return pl.pallas_call(
        _smerge_body,
        grid=(grid,),
        in_specs=[blk3, blk3],
        out_specs=blk2,
        out_shape=jax.ShapeDtypeStruct((rows, D), f32),
    )(mpart, spart)


def _comb_body(w_ref, wc_ref, bc_ref, o_ref):
    o_ref[...] = (jnp.dot(w_ref[...], wc_ref[...], preferred_element_type=f32)
                  + bc_ref[...])


def _comb_proj(w, W_comb, b_comb):
    be = 4000
    grid = E // be
    return pl.pallas_call(
        _comb_body,
        grid=(grid,),
        in_specs=[pl.BlockSpec((be, D), lambda i: (i, 0)),
                  pl.BlockSpec((D, D), lambda i: (0, 0)),
                  pl.BlockSpec((1, D), lambda i: (0, 0))],
        out_specs=pl.BlockSpec((be, D), lambda i: (i, 0)),
        out_shape=jax.ShapeDtypeStruct((E, D), f32),
    )(w, W_comb, b_comb.reshape(1, D))


def _sc_mesh():
    return plsc.VectorSubcoreMesh(core_axis_name="c", subcore_axis_name="s",
                                  num_cores=NC, num_subcores=NS)


def _wid():
    return lax.axis_index("s") * NC + lax.axis_index("c")


def _al8(x):
    return pl.multiple_of(x, 8)


_SC_PARAMS = pltpu.CompilerParams(needs_layout_passes=False)

G2 = 80
NCH2 = EPW // G2


def _k2a_body(q_hbm, k_hbm, re_hbm, ni_hbm, nj_hbm,
              l_hbm,
              ni_v, nj_v,
              q0, k0, re0, q1, k1, re1, l0, l1,
              gs0, gs1, ls0, ls1):
    wid = _wid()
    base0 = wid * EPW
    iota = lax.iota(i32, 16)
    pltpu.sync_copy(ni_hbm.at[pl.ds(_al8(base0), EPW)], ni_v)
    pltpu.sync_copy(nj_hbm.at[pl.ds(_al8(base0), EPW)], nj_v)
    bufs = ((q0, k0, re0, l0, gs0, ls0), (q1, k1, re1, l1, gs1, ls1))

    def issue(c, b):
        qb, kb, rb, _, gs, _ = bufs[b]
        off = _al8(c * G2)
        pltpu.async_copy(q_hbm.at[ni_v.at[pl.ds(off, G2)]], qb, gs)
        pltpu.async_copy(k_hbm.at[nj_v.at[pl.ds(off, G2)]], kb, gs)
        pltpu.async_copy(re_hbm.at[pl.ds(_al8(base0 + c * G2), G2)], rb, gs)

    def wait_gather(b):
        qb, kb, rb, _, gs, _ = bufs[b]
        pltpu.make_async_copy(q_hbm.at[ni_v.at[pl.ds(0, G2)]], qb, gs).wait()
        pltpu.make_async_copy(k_hbm.at[nj_v.at[pl.ds(0, G2)]], kb, gs).wait()
        pltpu.make_async_copy(re_hbm.at[pl.ds(0, G2)], rb, gs).wait()

    def wait_lwrite(b):
        _, _, _, lb, _, ls = bufs[b]
        pltpu.make_async_copy(lb, l_hbm.at[pl.ds(0, G2 * H)], ls).wait()

    def compute(c, b):
        qb, kb, rb, lb, _, ls = bufs[b]

        def edge(ep, c2):
            lacc = jnp.zeros((16,), f32)
            for half in range(2):
                e = 2 * ep + half
                for h in range(H):
                    qv = qb[e, pl.ds(h * DH, DH)]
                    kv = kb[e, pl.ds(h * DH, DH)]
                    rv = rb[e, pl.ds(h * DH, DH)]
                    s = jnp.sum(qv * kv * rv)
                    lacc = jnp.where(iota == (half * H + h),
                                     jnp.broadcast_to(s, (16,)), lacc)
            lb[pl.ds(_al8(ep * 16), 16)] = lacc
            return c2
        lax.fori_loop(0, G2 // 2, edge, 0)
        pltpu.async_copy(
            lb, l_hbm.at[pl.ds(_al8((base0 + c * G2) * H), G2 * H)], ls)

    issue(0, 0)
    issue(1, 1)

    def sup(s2, carry):
        for b in range(2):
            c = 2 * s2 + b
            wait_gather(b)

            @pl.when(c >= 2)
            def _():
                wait_lwrite(b)
            compute(c, b)

            @pl.when(c + 2 < NCH2)
            def _():
                issue(c + 2, b)
        return carry
    lax.fori_loop(0, (NCH2 - 1) // 2, sup, 0)
    wait_gather(0)
    wait_lwrite(0)
    compute(NCH2 - 1, 0)
    wait_lwrite(1)
    wait_lwrite(0)


def _k2a_call(q, k, re, ni, nj):
    return pl.kernel(
        _k2a_body,
        mesh=_sc_mesh(),
        compiler_params=_SC_PARAMS,
        out_type=jax.ShapeDtypeStruct((E * H,), f32),
        scratch_types=[
            pltpu.VMEM((EPW,), i32),
            pltpu.VMEM((EPW,), i32),
            pltpu.VMEM((G2, D), f32),
            pltpu.VMEM((G2, D), f32),
            pltpu.VMEM((G2, D), f32),
            pltpu.VMEM((G2, D), f32),
            pltpu.VMEM((G2, D), f32),
            pltpu.VMEM((G2, D), f32),
            pltpu.VMEM((G2 * H,), f32),
            pltpu.VMEM((G2 * H,), f32),
            pltpu.SemaphoreType.DMA,
            pltpu.SemaphoreType.DMA,
            pltpu.SemaphoreType.DMA,
            pltpu.SemaphoreType.DMA,
        ],
    )(q, k, re, ni, nj)


B2B = 2000
NBLK2B = EPW // B2B


def _k2b_body(l_hbm, ni_hbm, neg_hbm,
              mpart_hbm,
              ni_v, l_v, acc_v, sem):
    wid = _wid()
    base0 = wid * EPW
    pltpu.sync_copy(neg_hbm, acc_v)
    iota = lax.iota(i32, 16)

    def block(bi, carry):
        base = _al8(base0 + bi * B2B)
        pltpu.sync_copy(ni_hbm.at[pl.ds(base, B2B)], ni_v)
        pltpu.sync_copy(l_hbm.at[pl.ds(_al8(base * H), B2B * H)], l_v)

        def grp(g, c2):
            ni16 = ni_v[pl.ds(g * 16, 16)]
            el = g * 16 + iota
            for h in range(H):
                f = ni16 * H + h
                arow = lax.shift_right_logical(f, 7)
                acol = jnp.bitwise_and(f, 127)
                val = plsc.load_gather(l_v, [el * H + h])
                cur = plsc.load_gather(acc_v, [arow, acol])
                m0 = val > cur

                def cond(m):
                    return jnp.any(m)

                def body(m):
                    plsc.store_scatter(acc_v, [arow, acol], val, mask=m)
                    cur2 = plsc.load_gather(acc_v, [arow, acol])
                    return val > cur2
                lax.while_loop(cond, body, m0)
            return c2
        lax.fori_loop(0, B2B // 16, grp, 0)
        return carry
    lax.fori_loop(0, NBLK2B, block, 0)
    pltpu.sync_copy(acc_v, mpart_hbm.at[wid])


def _k2b_call(l, ni, neg):
    return pl.kernel(
        _k2b_body,
        mesh=_sc_mesh(),
        compiler_params=_SC_PARAMS,
        out_type=jax.ShapeDtypeStruct((NW, N2 * H // D, D), f32),
        scratch_types=[
            pltpu.VMEM((B2B,), i32),
            pltpu.VMEM((B2B * H,), f32),
            pltpu.VMEM((N2 * H // D, D), f32),
            pltpu.SemaphoreType.DMA,
        ],
    )(l, ni, neg)


B5 = 400
NBLK5 = EPW // B5
NHALF = N2 * H // D // 2


def _k5_body(half, l_hbm, ni_hbm, mpart_hbm, zero_hbm,
             spart_hbm,
             ni_v, l_v, m_acc, s_acc, sem):
    wid = _wid()
    base0 = wid * EPW
    iota = lax.iota(i32, 16)
    fbase = half * (NHALF * D)

    pltpu.sync_copy(mpart_hbm.at[wid, pl.ds(half * NHALF, NHALF)], m_acc)
    pltpu.sync_copy(zero_hbm, s_acc)

    def block(bi, carry):
        base = _al8(base0 + bi * B5)
        pltpu.sync_copy(ni_hbm.at[pl.ds(base, B5)], ni_v)
        pltpu.sync_copy(l_hbm.at[pl.ds(_al8(base * H), B5 * H)], l_v)

        def grp(g, c2):
            ni16 = ni_v[pl.ds(g * 16, 16)]
            el = g * 16 + iota
            for h in range(H):
                f = ni16 * H + h - fbase
                valid = jnp.logical_and(f >= 0, f < NHALF * D)
                fc = jnp.clip(f, 0, NHALF * D - 1)
                arow = lax.shift_right_logical(fc, 7)
                acol = jnp.bitwise_and(fc, 127)
                lv = plsc.load_gather(l_v, [el * H + h])
                mv = plsc.load_gather(m_acc, [arow, acol])
                ev = jnp.exp(lv - mv)
                plsc.addupdate_scatter(s_acc, [arow, acol], ev, mask=valid)
            return c2
        lax.fori_loop(0, B5 // 16, grp, 0)
        return carry
    lax.fori_loop(0, NBLK5, block, 0)
    pltpu.sync_copy(s_acc, spart_hbm.at[wid])


def _k5_call(half, l, ni, mpart, zero):
    return pl.kernel(
        functools.partial(_k5_body, half),
        mesh=_sc_mesh(),
        compiler_params=_SC_PARAMS,
        out_type=jax.ShapeDtypeStruct((NW, NHALF, D), f32),
        scratch_types=[
            pltpu.VMEM((B5,), i32),
            pltpu.VMEM((B5 * H,), f32),
            pltpu.VMEM((NHALF, D), f32),
            pltpu.VMEM((NHALF, D), f32),
            pltpu.SemaphoreType.DMA,
        ],
    )(l, ni, mpart, zero)


G7 = 80
NCH7 = EPW // G7
GROWS = N2 * H // D


def _k7_body(g_hbm, v_hbm, l_hbm, ni_hbm, nj_hbm,
             w_hbm,
             g_acc, ni0, nj0, ni1, nj1, v0, l0, v1, l1, w0, w1,
             gs0, gs1, ws0, ws1):
    wid = _wid()
    base0 = wid * EPW
    iota = lax.iota(i32, 16)
    cols = jnp.bitwise_and(iota, 7)
    pltpu.sync_copy(g_hbm, g_acc)
    bufs = ((ni0, nj0, v0, l0, w0, gs0, ws0), (ni1, nj1, v1, l1, w1, gs1, ws1))

    def issue(c, b):
        nib, njb, vb, lb, _, gs, _ = bufs[b]
        base = _al8(base0 + c * G7)
        pltpu.async_copy(ni_hbm.at[pl.ds(base, G7)], nib, gs)
        pltpu.async_copy(nj_hbm.at[pl.ds(base, G7)], njb, gs)
        pltpu.async_copy(l_hbm.at[pl.ds(_al8(base * H), G7 * H)], lb, gs)

    def issue2(b):
        nib, njb, vb, _, _, gs, _ = bufs[b]
        pltpu.async_copy(v_hbm.at[njb], vb, gs)

    def wait3(b):
        nib, njb, vb, lb, _, gs, _ = bufs[b]
        pltpu.make_async_copy(ni_hbm.at[pl.ds(0, G7)], nib, gs).wait()
        pltpu.make_async_copy(ni_hbm.at[pl.ds(0, G7)], njb, gs).wait()
        pltpu.make_async_copy(l_hbm.at[pl.ds(0, G7 * H)], lb, gs).wait()

    def wait_v(b):
        nib, njb, vb, _, _, gs, _ = bufs[b]
        pltpu.make_async_copy(v_hbm.at[njb], vb, gs).wait()

    def wait_w(b):
        _, _, _, _, wb, _, ws = bufs[b]
        pltpu.make_async_copy(wb, w_hbm.at[pl.ds(0, G7)], ws).wait()

    def compute(c, b):
        nib, njb, vb, lb, wb, _, ws = bufs[b]

        def grp(g, c2):
            ni16 = nib[pl.ds(_al8(g * 16), 16)]
            for p in range(8):
                na = ni16[2 * p]
                nb = ni16[2 * p + 1]
                gi = jnp.where(iota < 8, jnp.broadcast_to(na * H, (16,)),
                               jnp.broadcast_to(nb * H, (16,))) + cols
                grow = lax.shift_right_logical(gi, 7)
                gcol = jnp.bitwise_and(gi, 127)
                gv = plsc.load_gather(g_acc, [grow, gcol])
                lv = lb[pl.ds(_al8((g * 8 + p) * 16), 16)]
                ad = jnp.exp(lv - gv)
                for half in range(2):
                    e = (g * 8 + p) * 2 + half
                    for h in range(H):
                        a_s = ad[half * H + h]
                        wb[e, pl.ds(h * DH, DH)] = vb[e, pl.ds(h * DH, DH)] * a_s
            return c2
        lax.fori_loop(0, G7 // 16, grp, 0)
        pltpu.async_copy(wb, w_hbm.at[pl.ds(_al8(base0 + c * G7), G7)], ws)

    issue(0, 0)
    issue(1, 1)

    def sup(s2, carry):
        for b in range(2):
            c = 2 * s2 + b
            wait3(b)
            issue2(b)
            wait_v(b)

            @pl.when(c >= 2)
            def _():
                wait_w(b)
            compute(c, b)

            @pl.when(c + 2 < NCH7)
            def _():
                issue(c + 2, b)
        return carry
    lax.fori_loop(0, (NCH7 - 1) // 2, sup, 0)
    wait3(0)
    issue2(0)
    wait_v(0)
    wait_w(0)
    compute(NCH7 - 1, 0)
    wait_w(1)
    wait_w(0)


def _k7_call(g, v, l, ni, nj):
    return pl.kernel(
        _k7_body,
        mesh=_sc_mesh(),
        compiler_params=_SC_PARAMS,
        out_type=jax.ShapeDtypeStruct((E, D), f32),
        scratch_types=[
            pltpu.VMEM((GROWS, D), f32),
            pltpu.VMEM((G7,), i32),
            pltpu.VMEM((G7,), i32),
            pltpu.VMEM((G7,), i32),
            pltpu.VMEM((G7,), i32),
            pltpu.VMEM((G7, D), f32),
            pltpu.VMEM((G7 * H,), f32),
            pltpu.VMEM((G7, D), f32),
            pltpu.VMEM((G7 * H,), f32),
            pltpu.VMEM((G7, D), f32),
            pltpu.VMEM((G7, D), f32),
            pltpu.SemaphoreType.DMA,
            pltpu.SemaphoreType.DMA,
            pltpu.SemaphoreType.DMA,
            pltpu.SemaphoreType.DMA,
        ],
    )(g, v, l, ni, nj)


def kernel(h, t_ij, edge_index, W_q, b_q, W_k, b_k, W_v1, b_v1, W_v2, b_v2,
           W_re, b_re, W_comb, b_comb):
    n_j = edge_index[0].astype(i32)
    n_i = edge_index[1].astype(i32)

    q, k, v = _node_proj(h, W_q, b_q, W_k, b_k, W_v1, b_v1, W_v2, b_v2)
    re = _re_proj(t_ij, W_re, b_re)

    l = _k2a_call(q, k, re, n_i, n_j)

    neg = jnp.full((N2 * H // D, D), NEG, f32)
    mpart = _k2b_call(l, n_i, neg)

    zero = jnp.zeros((NHALF, D), f32)
    spart0 = _k5_call(0, l, n_i, mpart, zero)
    spart1 = _k5_call(1, l, n_i, mpart, zero)

    spart = jnp.concatenate([spart0, spart1], axis=1)
    g = _softmax_merge(mpart, spart)

    w = _k7_call(g, v, l, n_i, n_j)

    return _comb_proj(w, W_comb, b_comb)


# speedup vs baseline: 7.6554x; 1.1805x over previous
import functools
import math

import jax
import jax.numpy as jnp
from jax import lax
from jax.experimental import pallas as pl
from jax.experimental.pallas import tpu as pltpu
from jax.experimental.pallas import tpu_sc as plsc

N = 10000
E = 320000
D = 128
H = 8
DH = 16
ED = 16
NORM = 1.0 / math.sqrt(DH)

NC = 2
NS = 16
NW = NC * NS
EPW = E // NW
N2 = 10240
NEG = -3.4028235e38

f32 = jnp.float32
i32 = jnp.int32


def _proj_body(h_ref, wq_ref, bq_ref, wk_ref, bk_ref, wv1_ref, bv1_ref,
               wv2_ref, bv2_ref, q_ref, k_ref, v_ref):
    hb = h_ref[...]
    q_ref[...] = jnp.dot(hb, wq_ref[...], preferred_element_type=f32) + bq_ref[...]
    k_ref[...] = jnp.dot(hb, wk_ref[...], preferred_element_type=f32) + bk_ref[...]
    t = jnp.dot(hb, wv1_ref[...], preferred_element_type=f32) + bv1_ref[...]
    t = t * jax.nn.sigmoid(t)
    v_ref[...] = jnp.dot(t, wv2_ref[...], preferred_element_type=f32) + bv2_ref[...]


def _node_proj(h, W_q, b_q, W_k, b_k, W_v1, b_v1, W_v2, b_v2):
    bn = 2000
    grid = N // bn
    blk = pl.BlockSpec((bn, D), lambda i: (i, 0))
    wblk = pl.BlockSpec((D, D), lambda i: (0, 0))
    bblk = pl.BlockSpec((1, D), lambda i: (0, 0))
    out = jax.ShapeDtypeStruct((N, D), f32)
    return pl.pallas_call(
        _proj_body,
        grid=(grid,),
        in_specs=[blk, wblk, bblk, wblk, bblk, wblk, bblk, wblk, bblk],
        out_specs=[blk, blk, blk],
        out_shape=[out, out, out],
    )(h, W_q, b_q.reshape(1, D), W_k, b_k.reshape(1, D),
      W_v1, b_v1.reshape(1, D), W_v2, b_v2.reshape(1, D))


def _re_body(t_ref, w_ref, b_ref, o_ref):
    t = jnp.dot(t_ref[...], w_ref[...], preferred_element_type=f32) + b_ref[...]
    o_ref[...] = t * jax.nn.sigmoid(t)


def _re_proj(t_ij, W_re, b_re):
    be = 4000
    grid = E // be
    return pl.pallas_call(
        _re_body,
        grid=(grid,),
        in_specs=[pl.BlockSpec((be, ED), lambda i: (i, 0)),
                  pl.BlockSpec((ED, D), lambda i: (0, 0)),
                  pl.BlockSpec((1, D), lambda i: (0, 0))],
        out_specs=pl.BlockSpec((be, D), lambda i: (i, 0)),
        out_shape=jax.ShapeDtypeStruct((E, D), f32),
    )(t_ij, W_re, b_re.reshape(1, D))


def _smerge_body(mp_ref, sp_ref, g_ref):
    mp = mp_ref[...]
    sp = sp_ref[...]
    m = jnp.max(mp, axis=0)
    s = jnp.sum(sp * jnp.exp(mp - m[None]), axis=0)
    g_ref[...] = m - math.log(NORM) + jnp.log(s + 1e-16)


def _softmax_merge(mpart, spart):
    rows = N2 * H // D
    bn = 160
    grid = rows // bn
    blk3 = pl.BlockSpec((NW, bn, D), lambda i: (0, i, 0))
    blk2 = pl.BlockSpec((bn, D), lambda i: (i, 0))
    return pl.pallas_call(
        _smerge_body,
        grid=(grid,),
        in_specs=[blk3, blk3],
        out_specs=blk2,
        out_shape=jax.ShapeDtypeStruct((rows, D), f32),
    )(mpart, spart)


def _comb_body(w_ref, wc_ref, bc_ref, o_ref):
    o_ref[...] = (jnp.dot(w_ref[...], wc_ref[...], preferred_element_type=f32)
                  + bc_ref[...])


def _comb_proj(w, W_comb, b_comb):
    be = 4000
    grid = E // be
    return pl.pallas_call(
        _comb_body,
        grid=(grid,),
        in_specs=[pl.BlockSpec((be, D), lambda i: (i, 0)),
                  pl.BlockSpec((D, D), lambda i: (0, 0)),
                  pl.BlockSpec((1, D), lambda i: (0, 0))],
        out_specs=pl.BlockSpec((be, D), lambda i: (i, 0)),
        out_shape=jax.ShapeDtypeStruct((E, D), f32),
    )(w, W_comb, b_comb.reshape(1, D))


def _sc_mesh():
    return plsc.VectorSubcoreMesh(core_axis_name="c", subcore_axis_name="s",
                                  num_cores=NC, num_subcores=NS)


def _wid():
    return lax.axis_index("s") * NC + lax.axis_index("c")


def _al8(x):
    return pl.multiple_of(x, 8)


def _vtake(x, idx):
    return lax.gather(
        x, idx[:, None],
        lax.GatherDimensionNumbers(offset_dims=(), collapsed_slice_dims=(0,),
                                   start_index_map=(0,)),
        (1,), mode=lax.GatherScatterMode.PROMISE_IN_BOUNDS)


_SC_PARAMS = pltpu.CompilerParams(needs_layout_passes=False)

G2 = 80
NCH2 = EPW // G2


def _k2a_body(q_hbm, k_hbm, re_hbm, ni_hbm, nj_hbm,
              l_hbm,
              ni_v, nj_v,
              q0, k0, re0, q1, k1, re1, l0, l1,
              gs0, gs1, ls0, ls1):
    wid = _wid()
    base0 = wid * EPW
    iota = lax.iota(i32, 16)
    pltpu.sync_copy(ni_hbm.at[pl.ds(_al8(base0), EPW)], ni_v)
    pltpu.sync_copy(nj_hbm.at[pl.ds(_al8(base0), EPW)], nj_v)
    bufs = ((q0, k0, re0, l0, gs0, ls0), (q1, k1, re1, l1, gs1, ls1))

    def issue(c, b):
        qb, kb, rb, _, gs, _ = bufs[b]
        off = _al8(c * G2)
        pltpu.async_copy(q_hbm.at[ni_v.at[pl.ds(off, G2)]], qb, gs)
        pltpu.async_copy(k_hbm.at[nj_v.at[pl.ds(off, G2)]], kb, gs)
        pltpu.async_copy(re_hbm.at[pl.ds(_al8(base0 + c * G2), G2)], rb, gs)

    def wait_gather(b):
        qb, kb, rb, _, gs, _ = bufs[b]
        pltpu.make_async_copy(q_hbm.at[ni_v.at[pl.ds(0, G2)]], qb, gs).wait()
        pltpu.make_async_copy(k_hbm.at[nj_v.at[pl.ds(0, G2)]], kb, gs).wait()
        pltpu.make_async_copy(re_hbm.at[pl.ds(0, G2)], rb, gs).wait()

    def wait_lwrite(b):
        _, _, _, lb, _, ls = bufs[b]
        pltpu.make_async_copy(lb, l_hbm.at[pl.ds(0, G2 * H)], ls).wait()

    def compute(c, b):
        qb, kb, rb, lb, _, ls = bufs[b]

        def edge(ep, c2):
            lacc = jnp.zeros((16,), f32)
            for half in range(2):
                e = 2 * ep + half
                for h in range(H):
                    qv = qb[e, pl.ds(h * DH, DH)]
                    kv = kb[e, pl.ds(h * DH, DH)]
                    rv = rb[e, pl.ds(h * DH, DH)]
                    s = jnp.sum(qv * kv * rv)
                    lacc = jnp.where(iota == (half * H + h),
                                     jnp.broadcast_to(s, (16,)), lacc)
            lb[pl.ds(_al8(ep * 16), 16)] = lacc
            return c2
        lax.fori_loop(0, G2 // 2, edge, 0)
        pltpu.async_copy(
            lb, l_hbm.at[pl.ds(_al8((base0 + c * G2) * H), G2 * H)], ls)

    issue(0, 0)
    issue(1, 1)

    def sup(s2, carry):
        for b in range(2):
            c = 2 * s2 + b
            wait_gather(b)

            @pl.when(c >= 2)
            def _():
                wait_lwrite(b)
            compute(c, b)

            @pl.when(c + 2 < NCH2)
            def _():
                issue(c + 2, b)
        return carry
    lax.fori_loop(0, (NCH2 - 1) // 2, sup, 0)
    wait_gather(0)
    wait_lwrite(0)
    compute(NCH2 - 1, 0)
    wait_lwrite(1)
    wait_lwrite(0)


def _k2a_call(q, k, re, ni, nj):
    return pl.kernel(
        _k2a_body,
        mesh=_sc_mesh(),
        compiler_params=_SC_PARAMS,
        out_type=jax.ShapeDtypeStruct((E * H,), f32),
        scratch_types=[
            pltpu.VMEM((EPW,), i32),
            pltpu.VMEM((EPW,), i32),
            pltpu.VMEM((G2, D), f32),
            pltpu.VMEM((G2, D), f32),
            pltpu.VMEM((G2, D), f32),
            pltpu.VMEM((G2, D), f32),
            pltpu.VMEM((G2, D), f32),
            pltpu.VMEM((G2, D), f32),
            pltpu.VMEM((G2 * H,), f32),
            pltpu.VMEM((G2 * H,), f32),
            pltpu.SemaphoreType.DMA,
            pltpu.SemaphoreType.DMA,
            pltpu.SemaphoreType.DMA,
            pltpu.SemaphoreType.DMA,
        ],
    )(q, k, re, ni, nj)


B2B = 2000
NBLK2B = EPW // B2B


def _k2b_body(l_hbm, ni_hbm, neg_hbm,
              mpart_hbm,
              ni_v, l_v, acc_v, sem):
    wid = _wid()
    base0 = wid * EPW
    pltpu.sync_copy(neg_hbm, acc_v)
    iota = lax.iota(i32, 16)

    def block(bi, carry):
        base = _al8(base0 + bi * B2B)
        pltpu.sync_copy(ni_hbm.at[pl.ds(base, B2B)], ni_v)
        pltpu.sync_copy(l_hbm.at[pl.ds(_al8(base * H), B2B * H)], l_v)

        def grp(g, c2):
            ni16 = ni_v[pl.ds(g * 16, 16)]
            el = g * 16 + iota
            srt, _ = plsc.sort_key_val(ni16, ni16)
            nxt = _vtake(srt, jnp.bitwise_and(iota + 1, 15))
            hasdup = jnp.any(jnp.logical_and(srt == nxt, iota < 15))

            def upd(verify):
                for h in range(H):
                    f = ni16 * H + h
                    arow = lax.shift_right_logical(f, 7)
                    acol = jnp.bitwise_and(f, 127)
                    val = plsc.load_gather(l_v, [el * H + h])
                    cur = plsc.load_gather(acc_v, [arow, acol])
                    m0 = val > cur
                    if not verify:
                        plsc.store_scatter(acc_v, [arow, acol], val, mask=m0)
                    else:
                        def cond(m):
                            return jnp.any(m)

                        def body(m):
                            plsc.store_scatter(acc_v, [arow, acol], val, mask=m)
                            cur2 = plsc.load_gather(acc_v, [arow, acol])
                            return val > cur2
                        lax.while_loop(cond, body, m0)

            lax.cond(hasdup, lambda: upd(True), lambda: upd(False))
            return c2
        lax.fori_loop(0, B2B // 16, grp, 0)
        return carry
    lax.fori_loop(0, NBLK2B, block, 0)
    pltpu.sync_copy(acc_v, mpart_hbm.at[wid])


def _k2b_call(l, ni, neg):
    return pl.kernel(
        _k2b_body,
        mesh=_sc_mesh(),
        compiler_params=_SC_PARAMS,
        out_type=jax.ShapeDtypeStruct((NW, N2 * H // D, D), f32),
        scratch_types=[
            pltpu.VMEM((B2B,), i32),
            pltpu.VMEM((B2B * H,), f32),
            pltpu.VMEM((N2 * H // D, D), f32),
            pltpu.SemaphoreType.DMA,
        ],
    )(l, ni, neg)


B5 = 2000
NBLK5 = EPW // B5
NHALF = N2 * H // D // 2


def _k5_body(half, l_hbm, ni_hbm, mpart_hbm, zero_hbm,
             spart_hbm,
             ni_v, l_v, m_acc, s_acc, sem):
    wid = _wid()
    base0 = wid * EPW
    iota = lax.iota(i32, 16)
    fbase = half * (NHALF * D)

    pltpu.sync_copy(mpart_hbm.at[wid, pl.ds(half * NHALF, NHALF)], m_acc)
    pltpu.sync_copy(zero_hbm, s_acc)

    def block(bi, carry):
        base = _al8(base0 + bi * B5)
        pltpu.sync_copy(ni_hbm.at[pl.ds(base, B5)], ni_v)
        pltpu.sync_copy(l_hbm.at[pl.ds(_al8(base * H), B5 * H)], l_v)

        def grp(g, c2):
            ni16 = ni_v[pl.ds(g * 16, 16)]
            el = g * 16 + iota
            for h in range(H):
                f = ni16 * H + h - fbase
                valid = jnp.logical_and(f >= 0, f < NHALF * D)
                fc = jnp.clip(f, 0, NHALF * D - 1)
                arow = lax.shift_right_logical(fc, 7)
                acol = jnp.bitwise_and(fc, 127)
                lv = plsc.load_gather(l_v, [el * H + h])
                mv = plsc.load_gather(m_acc, [arow, acol])
                ev = jnp.exp(lv - mv)
                plsc.addupdate_scatter(s_acc, [arow, acol], ev, mask=valid)
            return c2
        lax.fori_loop(0, B5 // 16, grp, 0)
        return carry
    lax.fori_loop(0, NBLK5, block, 0)
    pltpu.sync_copy(s_acc, spart_hbm.at[wid])


def _k5_call(half, l, ni, mpart, zero):
    return pl.kernel(
        functools.partial(_k5_body, half),
        mesh=_sc_mesh(),
        compiler_params=_SC_PARAMS,
        out_type=jax.ShapeDtypeStruct((NW, NHALF, D), f32),
        scratch_types=[
            pltpu.VMEM((B5,), i32),
            pltpu.VMEM((B5 * H,), f32),
            pltpu.VMEM((NHALF, D), f32),
            pltpu.VMEM((NHALF, D), f32),
            pltpu.SemaphoreType.DMA,
        ],
    )(l, ni, mpart, zero)


G7 = 80
NCH7 = EPW // G7
GROWS = N2 * H // D


def _k7_body(g_hbm, v_hbm, l_hbm, ni_hbm, nj_hbm,
             w_hbm,
             g_acc, ni0, nj0, ni1, nj1, v0, l0, v1, l1, w0, w1,
             gs0, gs1, ws0, ws1):
    wid = _wid()
    base0 = wid * EPW
    iota = lax.iota(i32, 16)
    cols = jnp.bitwise_and(iota, 7)
    pltpu.sync_copy(g_hbm, g_acc)
    bufs = ((ni0, nj0, v0, l0, w0, gs0, ws0), (ni1, nj1, v1, l1, w1, gs1, ws1))

    def issue(c, b):
        nib, njb, vb, lb, _, gs, _ = bufs[b]
        base = _al8(base0 + c * G7)
        pltpu.async_copy(ni_hbm.at[pl.ds(base, G7)], nib, gs)
        pltpu.async_copy(nj_hbm.at[pl.ds(base, G7)], njb, gs)
        pltpu.async_copy(l_hbm.at[pl.ds(_al8(base * H), G7 * H)], lb, gs)

    def issue2(b):
        nib, njb, vb, _, _, gs, _ = bufs[b]
        pltpu.async_copy(v_hbm.at[njb], vb, gs)

    def wait3(b):
        nib, njb, vb, lb, _, gs, _ = bufs[b]
        pltpu.make_async_copy(ni_hbm.at[pl.ds(0, G7)], nib, gs).wait()
        pltpu.make_async_copy(ni_hbm.at[pl.ds(0, G7)], njb, gs).wait()
        pltpu.make_async_copy(l_hbm.at[pl.ds(0, G7 * H)], lb, gs).wait()

    def wait_v(b):
        nib, njb, vb, _, _, gs, _ = bufs[b]
        pltpu.make_async_copy(v_hbm.at[njb], vb, gs).wait()

    def wait_w(b):
        _, _, _, _, wb, _, ws = bufs[b]
        pltpu.make_async_copy(wb, w_hbm.at[pl.ds(0, G7)], ws).wait()

    def compute(c, b):
        nib, njb, vb, lb, wb, _, ws = bufs[b]

        def grp(g, c2):
            ni16 = nib[pl.ds(_al8(g * 16), 16)]
            for p in range(8):
                na = ni16[2 * p]
                nb = ni16[2 * p + 1]
                gi = jnp.where(iota < 8, jnp.broadcast_to(na * H, (16,)),
                               jnp.broadcast_to(nb * H, (16,))) + cols
                grow = lax.shift_right_logical(gi, 7)
                gcol = jnp.bitwise_and(gi, 127)
                gv = plsc.load_gather(g_acc, [grow, gcol])
                lv = lb[pl.ds(_al8((g * 8 + p) * 16), 16)]
                ad = jnp.exp(lv - gv)
                for half in range(2):
                    e = (g * 8 + p) * 2 + half
                    for h in range(H):
                        a_s = ad[half * H + h]
                        wb[e, pl.ds(h * DH, DH)] = vb[e, pl.ds(h * DH, DH)] * a_s
            return c2
        lax.fori_loop(0, G7 // 16, grp, 0)
        pltpu.async_copy(wb, w_hbm.at[pl.ds(_al8(base0 + c * G7), G7)], ws)

    issue(0, 0)
    issue(1, 1)

    def sup(s2, carry):
        for b in range(2):
            c = 2 * s2 + b
            wait3(b)
            issue2(b)
            wait_v(b)

            @pl.when(c >= 2)
            def _():
                wait_w(b)
            compute(c, b)

            @pl.when(c + 2 < NCH7)
            def _():
                issue(c + 2, b)
        return carry
    lax.fori_loop(0, (NCH7 - 1) // 2, sup, 0)
    wait3(0)
    issue2(0)
    wait_v(0)
    wait_w(0)
    compute(NCH7 - 1, 0)
    wait_w(1)
    wait_w(0)


def _k7_call(g, v, l, ni, nj):
    return pl.kernel(
        _k7_body,
        mesh=_sc_mesh(),
        compiler_params=_SC_PARAMS,
        out_type=jax.ShapeDtypeStruct((E, D), f32),
        scratch_types=[
            pltpu.VMEM((GROWS, D), f32),
            pltpu.VMEM((G7,), i32),
            pltpu.VMEM((G7,), i32),
            pltpu.VMEM((G7,), i32),
            pltpu.VMEM((G7,), i32),
            pltpu.VMEM((G7, D), f32),
            pltpu.VMEM((G7 * H,), f32),
            pltpu.VMEM((G7, D), f32),
            pltpu.VMEM((G7 * H,), f32),
            pltpu.VMEM((G7, D), f32),
            pltpu.VMEM((G7, D), f32),
            pltpu.SemaphoreType.DMA,
            pltpu.SemaphoreType.DMA,
            pltpu.SemaphoreType.DMA,
            pltpu.SemaphoreType.DMA,
        ],
    )(g, v, l, ni, nj)


def kernel(h, t_ij, edge_index, W_q, b_q, W_k, b_k, W_v1, b_v1, W_v2, b_v2,
           W_re, b_re, W_comb, b_comb):
    n_j = edge_index[0].astype(i32)
    n_i = edge_index[1].astype(i32)

    q, k, v = _node_proj(h, W_q, b_q, W_k, b_k, W_v1, b_v1, W_v2, b_v2)
    re = _re_proj(t_ij, W_re, b_re)

    l = _k2a_call(q, k, re, n_i, n_j)

    neg = jnp.full((N2 * H // D, D), NEG, f32)
    mpart = _k2b_call(l, n_i, neg)

    zero = jnp.zeros((NHALF, D), f32)
    spart0 = _k5_call(0, l, n_i, mpart, zero)
    spart1 = _k5_call(1, l, n_i, mpart, zero)

    spart = jnp.concatenate([spart0, spart1], axis=1)
    g = _softmax_merge(mpart, spart)

    w = _k7_call(g, v, l, n_i, n_j)

    return _comb_proj(w, W_comb, b_comb)


# speedup vs baseline: 8.1108x; 1.0595x over previous
import functools
import math

import jax
import jax.numpy as jnp
from jax import lax
from jax.experimental import pallas as pl
from jax.experimental.pallas import tpu as pltpu
from jax.experimental.pallas import tpu_sc as plsc

N = 10000
E = 320000
D = 128
H = 8
DH = 16
ED = 16
NORM = 1.0 / math.sqrt(DH)

NC = 2
NS = 16
NW = NC * NS
EPW = E // NW
N2 = 10240
NEG = -3.4028235e38

f32 = jnp.float32
i32 = jnp.int32


def _proj_body(h_ref, wq_ref, bq_ref, wk_ref, bk_ref, wv1_ref, bv1_ref,
               wv2_ref, bv2_ref, q_ref, k_ref, v_ref):
    hb = h_ref[...]
    q_ref[...] = jnp.dot(hb, wq_ref[...], preferred_element_type=f32) + bq_ref[...]
    k_ref[...] = jnp.dot(hb, wk_ref[...], preferred_element_type=f32) + bk_ref[...]
    t = jnp.dot(hb, wv1_ref[...], preferred_element_type=f32) + bv1_ref[...]
    t = t * jax.nn.sigmoid(t)
    v_ref[...] = jnp.dot(t, wv2_ref[...], preferred_element_type=f32) + bv2_ref[...]


def _node_proj(h, W_q, b_q, W_k, b_k, W_v1, b_v1, W_v2, b_v2):
    bn = 2000
    grid = N // bn
    blk = pl.BlockSpec((bn, D), lambda i: (i, 0))
    wblk = pl.BlockSpec((D, D), lambda i: (0, 0))
    bblk = pl.BlockSpec((1, D), lambda i: (0, 0))
    out = jax.ShapeDtypeStruct((N, D), f32)
    return pl.pallas_call(
        _proj_body,
        grid=(grid,),
        in_specs=[blk, wblk, bblk, wblk, bblk, wblk, bblk, wblk, bblk],
        out_specs=[blk, blk, blk],
        out_shape=[out, out, out],
    )(h, W_q, b_q.reshape(1, D), W_k, b_k.reshape(1, D),
      W_v1, b_v1.reshape(1, D), W_v2, b_v2.reshape(1, D))


def _re_body(t_ref, w_ref, b_ref, o_ref):
    t = jnp.dot(t_ref[...], w_ref[...], preferred_element_type=f32) + b_ref[...]
    o_ref[...] = t * jax.nn.sigmoid(t)


def _re_proj(t_ij, W_re, b_re):
    be = 4000
    grid = E // be
    return pl.pallas_call(
        _re_body,
        grid=(grid,),
        in_specs=[pl.BlockSpec((be, ED), lambda i: (i, 0)),
                  pl.BlockSpec((ED, D), lambda i: (0, 0)),
                  pl.BlockSpec((1, D), lambda i: (0, 0))],
        out_specs=pl.BlockSpec((be, D), lambda i: (i, 0)),
        out_shape=jax.ShapeDtypeStruct((E, D), f32),
    )(t_ij, W_re, b_re.reshape(1, D))


def _smerge_body(mp_ref, sp_ref, g_ref):
    mp = mp_ref[...]
    sp = sp_ref[...]
    m = jnp.max(mp, axis=0)
    s = jnp.sum(sp * jnp.exp(mp - m[None]), axis=0)
    g_ref[...] = m - math.log(NORM) + jnp.log(s + 1e-16)


def _softmax_merge(mpart, spart):
    rows = N2 * H // D
    bn = 160
    grid = rows // bn
    blk3 = pl.BlockSpec((NW, bn, D), lambda i: (0, i, 0))
    blk2 = pl.BlockSpec((bn, D), lambda i: (i, 0))
    return pl.pallas_call(
        _smerge_body,
        grid=(grid,),
        in_specs=[blk3, blk3],
        out_specs=blk2,
        out_shape=jax.ShapeDtypeStruct((rows, D), f32),
    )(mpart, spart)


def _comb_body(w_ref, wc_ref, bc_ref, o_ref):
    o_ref[...] = (jnp.dot(w_ref[...], wc_ref[...], preferred_element_type=f32)
                  + bc_ref[...])


def _comb_proj(w, W_comb, b_comb):
    be = 4000
    grid = E // be
    return pl.pallas_call(
        _comb_body,
        grid=(grid,),
        in_specs=[pl.BlockSpec((be, D), lambda i: (i, 0)),
                  pl.BlockSpec((D, D), lambda i: (0, 0)),
                  pl.BlockSpec((1, D), lambda i: (0, 0))],
        out_specs=pl.BlockSpec((be, D), lambda i: (i, 0)),
        out_shape=jax.ShapeDtypeStruct((E, D), f32),
    )(w, W_comb, b_comb.reshape(1, D))


def _sc_mesh():
    return plsc.VectorSubcoreMesh(core_axis_name="c", subcore_axis_name="s",
                                  num_cores=NC, num_subcores=NS)


def _wid():
    return lax.axis_index("s") * NC + lax.axis_index("c")


def _al8(x):
    return pl.multiple_of(x, 8)


def _vtake(x, idx):
    return lax.gather(
        x, idx[:, None],
        lax.GatherDimensionNumbers(offset_dims=(), collapsed_slice_dims=(0,),
                                   start_index_map=(0,)),
        (1,), mode=lax.GatherScatterMode.PROMISE_IN_BOUNDS)


_SC_PARAMS = pltpu.CompilerParams(needs_layout_passes=False)

G2 = 80
NCH2 = EPW // G2


def _k2a_body(q_hbm, k_hbm, re_hbm, ni_hbm, nj_hbm,
              l_hbm,
              ni_v, nj_v,
              q0, k0, re0, q1, k1, re1, l0, l1,
              gs0, gs1, ls0, ls1):
    wid = _wid()
    base0 = wid * EPW
    iota = lax.iota(i32, 16)
    pltpu.sync_copy(ni_hbm.at[pl.ds(_al8(base0), EPW)], ni_v)
    pltpu.sync_copy(nj_hbm.at[pl.ds(_al8(base0), EPW)], nj_v)
    bufs = ((q0, k0, re0, l0, gs0, ls0), (q1, k1, re1, l1, gs1, ls1))

    def issue(c, b):
        qb, kb, rb, _, gs, _ = bufs[b]
        off = _al8(c * G2)
        pltpu.async_copy(q_hbm.at[ni_v.at[pl.ds(off, G2)]], qb, gs)
        pltpu.async_copy(k_hbm.at[nj_v.at[pl.ds(off, G2)]], kb, gs)
        pltpu.async_copy(re_hbm.at[pl.ds(_al8(base0 + c * G2), G2)], rb, gs)

    def wait_gather(b):
        qb, kb, rb, _, gs, _ = bufs[b]
        pltpu.make_async_copy(q_hbm.at[ni_v.at[pl.ds(0, G2)]], qb, gs).wait()
        pltpu.make_async_copy(k_hbm.at[nj_v.at[pl.ds(0, G2)]], kb, gs).wait()
        pltpu.make_async_copy(re_hbm.at[pl.ds(0, G2)], rb, gs).wait()

    def wait_lwrite(b):
        _, _, _, lb, _, ls = bufs[b]
        pltpu.make_async_copy(lb, l_hbm.at[pl.ds(0, G2 * H)], ls).wait()

    def compute(c, b):
        qb, kb, rb, lb, _, ls = bufs[b]

        def edge(ep, c2):
            lacc = jnp.zeros((16,), f32)
            for half in range(2):
                e = 2 * ep + half
                for h in range(H):
                    qv = qb[e, pl.ds(h * DH, DH)]
                    kv = kb[e, pl.ds(h * DH, DH)]
                    rv = rb[e, pl.ds(h * DH, DH)]
                    s = jnp.sum(qv * kv * rv)
                    lacc = jnp.where(iota == (half * H + h),
                                     jnp.broadcast_to(s, (16,)), lacc)
            lb[pl.ds(_al8(ep * 16), 16)] = lacc
            return c2
        lax.fori_loop(0, G2 // 2, edge, 0)
        pltpu.async_copy(
            lb, l_hbm.at[pl.ds(_al8((base0 + c * G2) * H), G2 * H)], ls)

    issue(0, 0)
    issue(1, 1)

    def sup(s2, carry):
        for b in range(2):
            c = 2 * s2 + b
            wait_gather(b)

            @pl.when(c >= 2)
            def _():
                wait_lwrite(b)
            compute(c, b)

            @pl.when(c + 2 < NCH2)
            def _():
                issue(c + 2, b)
        return carry
    lax.fori_loop(0, (NCH2 - 1) // 2, sup, 0)
    wait_gather(0)
    wait_lwrite(0)
    compute(NCH2 - 1, 0)
    wait_lwrite(1)
    wait_lwrite(0)


def _k2a_call(q, k, re, ni, nj):
    return pl.kernel(
        _k2a_body,
        mesh=_sc_mesh(),
        compiler_params=_SC_PARAMS,
        out_type=jax.ShapeDtypeStruct((E * H,), f32),
        scratch_types=[
            pltpu.VMEM((EPW,), i32),
            pltpu.VMEM((EPW,), i32),
            pltpu.VMEM((G2, D), f32),
            pltpu.VMEM((G2, D), f32),
            pltpu.VMEM((G2, D), f32),
            pltpu.VMEM((G2, D), f32),
            pltpu.VMEM((G2, D), f32),
            pltpu.VMEM((G2, D), f32),
            pltpu.VMEM((G2 * H,), f32),
            pltpu.VMEM((G2 * H,), f32),
            pltpu.SemaphoreType.DMA,
            pltpu.SemaphoreType.DMA,
            pltpu.SemaphoreType.DMA,
            pltpu.SemaphoreType.DMA,
        ],
    )(q, k, re, ni, nj)


B2B = 2000
NBLK2B = EPW // B2B


def _k2b_body(l_hbm, ni_hbm, neg_hbm,
              mpart_hbm,
              ni_v, l_v, acc_v, sem):
    wid = _wid()
    base0 = wid * EPW
    pltpu.sync_copy(neg_hbm, acc_v)
    iota = lax.iota(i32, 16)

    def block(bi, carry):
        base = _al8(base0 + bi * B2B)
        pltpu.sync_copy(ni_hbm.at[pl.ds(base, B2B)], ni_v)
        pltpu.sync_copy(l_hbm.at[pl.ds(_al8(base * H), B2B * H)], l_v)

        def grp(g, c2):
            ni16 = ni_v[pl.ds(g * 16, 16)]
            el = g * 16 + iota
            srt, _ = plsc.sort_key_val(ni16, ni16)
            nxt = _vtake(srt, jnp.bitwise_and(iota + 1, 15))
            hasdup = jnp.any(jnp.logical_and(srt == nxt, iota < 15))

            def upd(verify):
                for h in range(H):
                    f = ni16 * H + h
                    arow = lax.shift_right_logical(f, 7)
                    acol = jnp.bitwise_and(f, 127)
                    val = plsc.load_gather(l_v, [el * H + h])
                    cur = plsc.load_gather(acc_v, [arow, acol])
                    m0 = val > cur
                    if not verify:
                        plsc.store_scatter(acc_v, [arow, acol], val, mask=m0)
                    else:
                        def cond(m):
                            return jnp.any(m)

                        def body(m):
                            plsc.store_scatter(acc_v, [arow, acol], val, mask=m)
                            cur2 = plsc.load_gather(acc_v, [arow, acol])
                            return val > cur2
                        lax.while_loop(cond, body, m0)

            lax.cond(hasdup, lambda: upd(True), lambda: upd(False))
            return c2
        lax.fori_loop(0, B2B // 16, grp, 0)
        return carry
    lax.fori_loop(0, NBLK2B, block, 0)
    pltpu.sync_copy(acc_v, mpart_hbm.at[wid])


def _k2b_call(l, ni, neg):
    return pl.kernel(
        _k2b_body,
        mesh=_sc_mesh(),
        compiler_params=_SC_PARAMS,
        out_type=jax.ShapeDtypeStruct((NW, N2 * H // D, D), f32),
        scratch_types=[
            pltpu.VMEM((B2B,), i32),
            pltpu.VMEM((B2B * H,), f32),
            pltpu.VMEM((N2 * H // D, D), f32),
            pltpu.SemaphoreType.DMA,
        ],
    )(l, ni, neg)


B5 = 2000
NBLK5 = EPW // B5
NHALF = N2 * H // D // 2


def _k5_body(half, l_hbm, ni_hbm, mpart_hbm, zero_hbm,
             spart_hbm,
             ni_v, l_v, m_acc, s_acc, sem):
    wid = _wid()
    base0 = wid * EPW
    iota = lax.iota(i32, 16)
    fbase = half * (NHALF * D)

    pltpu.sync_copy(mpart_hbm.at[wid, pl.ds(half * NHALF, NHALF)], m_acc)
    pltpu.sync_copy(zero_hbm, s_acc)

    def block(bi, carry):
        base = _al8(base0 + bi * B5)
        pltpu.sync_copy(ni_hbm.at[pl.ds(base, B5)], ni_v)
        pltpu.sync_copy(l_hbm.at[pl.ds(_al8(base * H), B5 * H)], l_v)

        def grp(g, c2):
            ni16 = ni_v[pl.ds(g * 16, 16)]
            el = g * 16 + iota
            for h in range(H):
                f = ni16 * H + h - fbase
                valid = jnp.logical_and(f >= 0, f < NHALF * D)
                fc = jnp.clip(f, 0, NHALF * D - 1)
                arow = lax.shift_right_logical(fc, 7)
                acol = jnp.bitwise_and(fc, 127)
                lv = plsc.load_gather(l_v, [el * H + h])
                mv = plsc.load_gather(m_acc, [arow, acol])
                ev = jnp.exp(lv - mv)
                plsc.addupdate_scatter(s_acc, [arow, acol], ev, mask=valid)
            return c2
        lax.fori_loop(0, B5 // 16, grp, 0)
        return carry
    lax.fori_loop(0, NBLK5, block, 0)
    pltpu.sync_copy(s_acc, spart_hbm.at[wid])


def _k5_call(half, l, ni, mpart, zero):
    return pl.kernel(
        functools.partial(_k5_body, half),
        mesh=_sc_mesh(),
        compiler_params=_SC_PARAMS,
        out_type=jax.ShapeDtypeStruct((NW, NHALF, D), f32),
        scratch_types=[
            pltpu.VMEM((B5,), i32),
            pltpu.VMEM((B5 * H,), f32),
            pltpu.VMEM((NHALF, D), f32),
            pltpu.VMEM((NHALF, D), f32),
            pltpu.SemaphoreType.DMA,
        ],
    )(l, ni, mpart, zero)


G7 = 80
NCH7 = EPW // G7
GROWS = N2 * H // D


def _k7_body(g_hbm, v_hbm, l_hbm, ni_hbm, nj_hbm,
             w_hbm,
             g_acc, ni0, nj0, ni1, nj1, v0, l0, v1, l1, w0, w1,
             gs0, gs1, ws0, ws1):
    wid = _wid()
    base0 = wid * EPW
    iota = lax.iota(i32, 16)
    cols = jnp.bitwise_and(iota, 7)
    pltpu.sync_copy(g_hbm, g_acc)
    bufs = ((ni0, nj0, v0, l0, w0, gs0, ws0), (ni1, nj1, v1, l1, w1, gs1, ws1))

    def issue(c, b):
        nib, njb, vb, lb, _, gs, _ = bufs[b]
        base = _al8(base0 + c * G7)
        pltpu.async_copy(ni_hbm.at[pl.ds(base, G7)], nib, gs)
        pltpu.async_copy(nj_hbm.at[pl.ds(base, G7)], njb, gs)
        pltpu.async_copy(l_hbm.at[pl.ds(_al8(base * H), G7 * H)], lb, gs)

    def issue2(b):
        nib, njb, vb, _, _, gs, _ = bufs[b]
        pltpu.async_copy(v_hbm.at[njb], vb, gs)

    def wait3(b):
        nib, njb, vb, lb, _, gs, _ = bufs[b]
        pltpu.make_async_copy(ni_hbm.at[pl.ds(0, G7)], nib, gs).wait()
        pltpu.make_async_copy(ni_hbm.at[pl.ds(0, G7)], njb, gs).wait()
        pltpu.make_async_copy(l_hbm.at[pl.ds(0, G7 * H)], lb, gs).wait()

    def wait_v(b):
        nib, njb, vb, _, _, gs, _ = bufs[b]
        pltpu.make_async_copy(v_hbm.at[njb], vb, gs).wait()

    def wait_w(b):
        _, _, _, _, wb, _, ws = bufs[b]
        pltpu.make_async_copy(wb, w_hbm.at[pl.ds(0, G7)], ws).wait()

    def compute(c, b):
        nib, njb, vb, lb, wb, _, ws = bufs[b]

        def grp(g, c2):
            ni16 = nib[pl.ds(_al8(g * 16), 16)]
            for p in range(8):
                na = ni16[2 * p]
                nb = ni16[2 * p + 1]
                gi = jnp.where(iota < 8, jnp.broadcast_to(na * H, (16,)),
                               jnp.broadcast_to(nb * H, (16,))) + cols
                grow = lax.shift_right_logical(gi, 7)
                gcol = jnp.bitwise_and(gi, 127)
                gv = plsc.load_gather(g_acc, [grow, gcol])
                lv = lb[pl.ds(_al8((g * 8 + p) * 16), 16)]
                ad = jnp.exp(lv - gv)
                for half in range(2):
                    e = (g * 8 + p) * 2 + half
                    for h in range(H):
                        a_s = ad[half * H + h]
                        wb[e, pl.ds(h * DH, DH)] = vb[e, pl.ds(h * DH, DH)] * a_s
            return c2
        lax.fori_loop(0, G7 // 16, grp, 0)
        pltpu.async_copy(wb, w_hbm.at[pl.ds(_al8(base0 + c * G7), G7)], ws)

    issue(0, 0)
    issue(1, 1)
    wait3(0)
    issue2(0)

    def sup(s2, carry):
        for b in range(2):
            c = 2 * s2 + b
            wait_v(b)
            wait3(1 - b)
            issue2(1 - b)

            @pl.when(c >= 2)
            def _():
                wait_w(b)
            compute(c, b)

            @pl.when(c + 2 < NCH7)
            def _():
                issue(c + 2, b)
        return carry
    lax.fori_loop(0, (NCH7 - 1) // 2, sup, 0)
    wait_v(0)
    wait_w(0)
    compute(NCH7 - 1, 0)
    wait_w(1)
    wait_w(0)


def _k7_call(g, v, l, ni, nj):
    return pl.kernel(
        _k7_body,
        mesh=_sc_mesh(),
        compiler_params=_SC_PARAMS,
        out_type=jax.ShapeDtypeStruct((E, D), f32),
        scratch_types=[
            pltpu.VMEM((GROWS, D), f32),
            pltpu.VMEM((G7,), i32),
            pltpu.VMEM((G7,), i32),
            pltpu.VMEM((G7,), i32),
            pltpu.VMEM((G7,), i32),
            pltpu.VMEM((G7, D), f32),
            pltpu.VMEM((G7 * H,), f32),
            pltpu.VMEM((G7, D), f32),
            pltpu.VMEM((G7 * H,), f32),
            pltpu.VMEM((G7, D), f32),
            pltpu.VMEM((G7, D), f32),
            pltpu.SemaphoreType.DMA,
            pltpu.SemaphoreType.DMA,
            pltpu.SemaphoreType.DMA,
            pltpu.SemaphoreType.DMA,
        ],
    )(g, v, l, ni, nj)


def kernel(h, t_ij, edge_index, W_q, b_q, W_k, b_k, W_v1, b_v1, W_v2, b_v2,
           W_re, b_re, W_comb, b_comb):
    n_j = edge_index[0].astype(i32)
    n_i = edge_index[1].astype(i32)

    q, k, v = _node_proj(h, W_q, b_q, W_k, b_k, W_v1, b_v1, W_v2, b_v2)
    re = _re_proj(t_ij, W_re, b_re)

    l = _k2a_call(q, k, re, n_i, n_j)

    neg = jnp.full((N2 * H // D, D), NEG, f32)
    mpart = _k2b_call(l, n_i, neg)

    zero = jnp.zeros((NHALF, D), f32)
    spart0 = _k5_call(0, l, n_i, mpart, zero)
    spart1 = _k5_call(1, l, n_i, mpart, zero)

    spart = jnp.concatenate([spart0, spart1], axis=1)
    g = _softmax_merge(mpart, spart)

    w = _k7_call(g, v, l, n_i, n_j)

    return _comb_proj(w, W_comb, b_comb)
